# baseline (speedup 1.0000x reference)
import dataclasses
import functools

import jax
import jax.numpy as jnp
from jax import lax
from jax.experimental import pallas as pl
from jax.experimental.pallas import tpu as pltpu
from jax.experimental.pallas import tpu_sc as plsc

N = 10000
E = 320000
H = 128
ED = 16
B = 64
T = 2

NBLK = 1000
EBLK = 4000
CH = 64
CHB = 64
NW = 32
RPT = 624


import numpy as _np

_pi = _np.arange(H)
_pg, _pj = _pi // 32, _pi % 32
QPERM = _np.where(_pj % 2 == 0, 32 * _pg + _pj // 2, 32 * _pg + 16 + _pj // 2)


def _leaky(v):
    return jnp.where(v > 0, v, 0.01 * v)


def _elu(v):
    return jnp.where(v > 0, v, jnp.exp(v) - 1.0)


def _tc1a_body(x_ref, w1t_ref, b1_ref, wxt_ref, x0_ref, u_ref):
    x0 = _leaky(jnp.dot(x_ref[...], w1t_ref[...],
                        preferred_element_type=jnp.float32) + b1_ref[...])
    x0_ref[...] = x0
    u_ref[...] = jnp.dot(x0, wxt_ref[...], preferred_element_type=jnp.float32)


def _tc1b_body(ea_ref, wet_ref, v_ref):
    v_ref[...] = jnp.dot(ea_ref[...], wet_ref[...],
                         preferred_element_type=jnp.float32).astype(jnp.bfloat16)


def _tc2_body(sp_ref, x0_ref, w2t_ref, ndb_ref, wih_ref, whh_ref, bih_ref,
              bhh_ref, gwt_ref, asd_w_ref, xh_ref, xw_ref, asd_ref):
    sp = sp_ref[...]
    s = sp[0] + sp[1]
    h = _elu(jnp.dot(s, w2t_ref[...], preferred_element_type=jnp.float32)
             + ndb_ref[...])
    x0 = x0_ref[...]
    gi = jnp.dot(h, wih_ref[...], preferred_element_type=jnp.float32) + bih_ref[...]
    gh = jnp.dot(x0, whh_ref[...], preferred_element_type=jnp.float32) + bhh_ref[...]
    r = jax.nn.sigmoid(gi[:, :H] + gh[:, :H])
    z = jax.nn.sigmoid(gi[:, H:2 * H] + gh[:, H:2 * H])
    n = jnp.tanh(gi[:, 2 * H:] + r * gh[:, 2 * H:])
    xh = jax.nn.relu((1.0 - z) * n + z * x0)
    xh_ref[...] = xh
    xw = jnp.dot(xh, gwt_ref[...], preferred_element_type=jnp.float32)
    xw_ref[...] = xw
    asd_ref[...] = jnp.dot(xw, asd_w_ref[...], preferred_element_type=jnp.float32)


def _tc3_body(acc_ref, zp_ref, xh_ref, batch_ref, gatb_ref, wih_ref, whh_ref,
              bih_ref, bhh_ref, ginwt_ref, ginb_ref, lwih_ref, lwhh_ref,
              lbih_ref, lbhh_ref, l2wt_ref, l2b_ref, out_ref, pooled_ref):
    i = pl.program_id(0)
    acc = acc_ref[...]
    w = acc[0] + acc[1]
    zp = zp_ref[...][0]
    z = jnp.sum(zp, axis=0)[:, None]
    gat = w / (z + 1e-16) + gatb_ref[...]
    h2 = _elu(gat)
    xh = xh_ref[...]
    gi = jnp.dot(h2, wih_ref[...], preferred_element_type=jnp.float32) + bih_ref[...]
    gh = jnp.dot(xh, whh_ref[...], preferred_element_type=jnp.float32) + bhh_ref[...]
    r = jax.nn.sigmoid(gi[:, :H] + gh[:, :H])
    zz = jax.nn.sigmoid(gi[:, H:2 * H] + gh[:, H:2 * H])
    n = jnp.tanh(gi[:, 2 * H:] + r * gh[:, 2 * H:])
    acts = jax.nn.relu((1.0 - zz) * n + zz * xh)

    bb = batch_ref[...][0]
    seg = lax.broadcasted_iota(jnp.int32, (B, NBLK), 0)
    oh = (bb == seg).astype(jnp.float32)
    part = jnp.dot(oh, acts, preferred_element_type=jnp.float32)

    @pl.when(i == 0)
    def _():
        pooled_ref[...] = part

    @pl.when(i > 0)
    def _():
        pooled_ref[...] = pooled_ref[...] + part

    @pl.when(i == pl.num_programs(0) - 1)
    def _():
        pooled = pooled_ref[...]
        out = jax.nn.relu(pooled)
        for _ in range(T):
            ghh = _elu(jnp.dot(out + pooled, ginwt_ref[...],
                               preferred_element_type=jnp.float32) + ginb_ref[...])
            g = (jnp.dot(out, lwih_ref[...], preferred_element_type=jnp.float32)
                 + lbih_ref[...]
                 + jnp.dot(ghh, lwhh_ref[...], preferred_element_type=jnp.float32)
                 + lbhh_ref[...])
            ig = jax.nn.sigmoid(g[:, :H])
            fg = jax.nn.sigmoid(g[:, H:2 * H])
            gg = jnp.tanh(g[:, 2 * H:3 * H])
            og = jax.nn.sigmoid(g[:, 3 * H:])
            c2 = fg * ghh + ig * gg
            out = og * jnp.tanh(c2)
        out_ref[...] = (jnp.dot(out, l2wt_ref[...],
                                preferred_element_type=jnp.float32) + l2b_ref[...])


def _zero_rows(buf, nrows):
    zero16 = jnp.zeros((16,), jnp.float32)
    ncols = buf.shape[1]

    @pl.loop(0, nrows)
    def _(r):
        for j in range(ncols // 16):
            buf[r, pl.ds(j * 16, 16)] = zero16


def _zero_acc_slice(sid, zbuf, acc):
    base_r = sid * RPT
    zch = zbuf.shape[0]
    for off in range(0, RPT, zch):
        sz = min(zch, RPT - off)
        pltpu.sync_copy(zbuf.at[pl.ds(0, sz)], acc.at[pl.ds(base_r + off, sz)])

    @pl.when(sid == 15)
    def _():
        pltpu.sync_copy(zbuf.at[pl.ds(0, N - 16 * RPT)],
                        acc.at[pl.ds(16 * RPT, N - 16 * RPT)])


def _flush_acc_slice(sid, cid, acc, out_hbm):
    base_r = sid * RPT
    pltpu.sync_copy(acc.at[pl.ds(base_r, RPT)],
                    out_hbm.at[cid, pl.ds(base_r, RPT)])

    @pl.when(sid == 15)
    def _():
        pltpu.sync_copy(acc.at[pl.ds(16 * RPT, N - 16 * RPT)],
                        out_hbm.at[cid, pl.ds(16 * RPT, N - 16 * RPT)])


def _edge_pipeline(wid, nchunk, ch, src_hbm, dst_hbm, table_hbm,
                   src_idxs, dst_idxs, g_bufs, s_dst, s_g, s_sc,
                   issue_aux, wait_aux, compute, scat_bufs, acc):
    def prefetch(bn, c_next):
        ebase = c_next * ch
        pltpu.sync_copy(src_hbm.at[pl.ds(ebase, ch)], src_idxs[bn])
        pltpu.async_copy(table_hbm.at[src_idxs[bn]], g_bufs[bn], s_g.at[bn])
        pltpu.async_copy(dst_hbm.at[pl.ds(ebase, ch)], dst_idxs[bn],
                         s_dst.at[bn])
        issue_aux(bn, ebase)

    prefetch(0, wid)
    itmax = -(-nchunk // NW)
    kpair = (itmax + 2) // 2

    @pl.loop(0, kpair)
    def _(k):
        for h in (0, 1):
            it = k * 2 + h
            b, bn = h, 1 - h
            c_idx = wid + it * NW

            @pl.when((it >= 1) & (c_idx < nchunk + NW))
            def _():
                pltpu.make_async_copy(scat_bufs[bn], acc.at[dst_idxs[bn]],
                                      s_sc.at[bn]).wait()

            @pl.when(c_idx + NW < nchunk)
            def _():
                prefetch(bn, c_idx + NW)

            @pl.when(c_idx < nchunk)
            def _():
                pltpu.make_async_copy(dst_hbm.at[pl.ds(0, ch)], dst_idxs[b],
                                      s_dst.at[b]).wait()
                pltpu.make_async_copy(table_hbm.at[src_idxs[b]], g_bufs[b],
                                      s_g.at[b]).wait()
                wait_aux(b)
                compute(b)
                pltpu.async_copy(scat_bufs[b], acc.at[dst_idxs[b]],
                                 s_sc.at[b], add=True)


def _sc_a_body(src_hbm, dst_hbm, u_hbm, v_hbm, out_hbm,
               src_idx0, src_idx1, dst_idx0, dst_idx1, u0, u1, v0, v1,
               m0, m1, acc, s_dst, s_g, s_aux, s_sc):
    cid = lax.axis_index("c")
    sid = lax.axis_index("s")
    wid = sid * 2 + cid
    src_idxs, dst_idxs = (src_idx0, src_idx1), (dst_idx0, dst_idx1)
    u_bufs, v_bufs, m_bufs = (u0, u1), (v0, v1), (m0, m1)

    _zero_rows(m0, CH)
    _zero_acc_slice(sid, m0, acc)

    def issue_aux(bn, ebase):
        pltpu.async_copy(v_hbm.at[pl.ds(ebase, CH)], v_bufs[bn], s_aux.at[bn])

    def wait_aux(b):
        pltpu.make_async_copy(v_hbm.at[pl.ds(0, CH)], v_bufs[b],
                              s_aux.at[b]).wait()

    def compute(b):
        ub, vb, mb = u_bufs[b], v_bufs[b], m_bufs[b]

        @pl.loop(0, CH // 2)
        def _(q):
            r0 = q * 2
            for rr in range(2):
                r = r0 + rr
                for g in range(H // 32):
                    vv = plsc.bitcast(vb[r, pl.ds(g * 16, 16)], jnp.bfloat16)
                    va, vb2 = plsc.unpack(vv, format=plsc.PackFormat.INTERLEAVED,
                                          preferred_element_type=jnp.float32)
                    t = ub[r, pl.ds(g * 32, 16)] + va
                    mb[r, pl.ds(g * 32, 16)] = jnp.where(t > 0, t, 0.01 * t)
                    t2 = ub[r, pl.ds(g * 32 + 16, 16)] + vb2
                    mb[r, pl.ds(g * 32 + 16, 16)] = jnp.where(t2 > 0, t2,
                                                              0.01 * t2)

    plsc.subcore_barrier()
    _edge_pipeline(wid, E // CH, CH, src_hbm, dst_hbm, u_hbm,
                   src_idxs, dst_idxs, u_bufs, s_dst, s_g, s_sc,
                   issue_aux, wait_aux, compute, m_bufs, acc)
    plsc.subcore_barrier()
    _flush_acc_slice(sid, cid, acc, out_hbm)


def _sc_b_body(src_hbm, dst_hbm, xw_hbm, asd_hbm, out_hbm, outz_hbm,
               src_idx0, src_idx1, dst_idx0, dst_idx1, pb0, pb1, xw0, xw1,
               asd_buf, z_buf, acc, s_dst, s_g, s_sc):
    cid = lax.axis_index("c")
    sid = lax.axis_index("s")
    wid = sid * 2 + cid
    lane = lax.iota(jnp.int32, 16)
    zero16 = jnp.zeros((16,), jnp.float32)
    src_idxs, dst_idxs = (src_idx0, src_idx1), (dst_idx0, dst_idx1)
    p_bufs, xw_bufs = (pb0, pb1), (xw0, xw1)

    _zero_rows(xw0, CHB)
    _zero_acc_slice(sid, xw0, acc)

    @pl.loop(0, N // 16)
    def _(r):
        z_buf[pl.ds(r * 16, 16)] = zero16

    pltpu.sync_copy(asd_hbm, asd_buf)

    def issue_aux(bn, ebase):
        pass

    def wait_aux(b):
        pass

    def compute(b):
        xb, pb_ref = xw_bufs[b], p_bufs[b]

        @pl.loop(0, CHB // 16)
        def _(j):
            sl = pl.ds(j * 16, 16)
            s16 = src_idxs[b][sl]
            d16 = dst_idxs[b][sl]
            a_s = plsc.load_gather(asd_buf, [s16 * 2])
            a_d = plsc.load_gather(asd_buf, [d16 * 2 + 1])
            e = a_s + a_d
            e = jnp.where(e > 0, e, 0.01 * e)
            p = jnp.exp(e)
            pb_ref[sl] = p
            for kk in range(16):
                plsc.addupdate_scatter(z_buf, [d16], p, mask=lane == kk)

        @pl.loop(0, CHB // 4)
        def _(q):
            r0 = q * 4
            for rr in range(4):
                r = r0 + rr
                pb = plsc.load_gather(pb_ref, [jnp.full((16,), r, jnp.int32)])
                for j in range(H // 16):
                    sl = pl.ds(j * 16, 16)
                    xb[r, sl] = xb[r, sl] * pb

    plsc.subcore_barrier()
    _edge_pipeline(wid, E // CHB, CHB, src_hbm, dst_hbm, xw_hbm,
                   src_idxs, dst_idxs, xw_bufs, s_dst, s_g, s_sc,
                   issue_aux, wait_aux, compute, xw_bufs, acc)
    plsc.subcore_barrier()
    _flush_acc_slice(sid, cid, acc, out_hbm)
    pltpu.sync_copy(z_buf, outz_hbm.at[cid, sid])


def kernel(x, edge_index, edge_attr, batch, lin1_W, lin1_b, nd1_W, nd2_W,
           nd_bias, gru0_Wih, gru0_Whh, gru0_bih, gru0_bhh, gat_W, gat_asrc,
           gat_adst, gat_b, gru1_Wih, gru1_Whh, gru1_bih, gru1_bhh, gin_W,
           gin_b, lstm_Wih, lstm_Whh, lstm_bih, lstm_bhh, lin2_W, lin2_b):
    f32 = jnp.float32
    src = edge_index[0]
    dst = edge_index[1]

    w1t = lin1_W.T
    b1 = lin1_b.reshape(1, H)
    wxt = nd1_W[:, :H].T
    wet = nd1_W[:, H:].T[:, QPERM]
    x0, u = pl.pallas_call(
        _tc1a_body,
        grid=(N // NBLK,),
        in_specs=[
            pl.BlockSpec((NBLK, H), lambda i: (i, 0)),
            pl.BlockSpec((H, H), lambda i: (0, 0)),
            pl.BlockSpec((1, H), lambda i: (0, 0)),
            pl.BlockSpec((H, H), lambda i: (0, 0)),
        ],
        out_specs=[
            pl.BlockSpec((NBLK, H), lambda i: (i, 0)),
            pl.BlockSpec((NBLK, H), lambda i: (i, 0)),
        ],
        out_shape=[jax.ShapeDtypeStruct((N, H), f32),
                   jax.ShapeDtypeStruct((N, H), f32)],
    )(x, w1t, b1, wxt)

    v = pl.pallas_call(
        _tc1b_body,
        grid=(E // EBLK,),
        in_specs=[
            pl.BlockSpec((EBLK, ED), lambda i: (i, 0)),
            pl.BlockSpec((ED, H), lambda i: (0, 0)),
        ],
        out_specs=pl.BlockSpec((EBLK, H), lambda i: (i, 0)),
        out_shape=jax.ShapeDtypeStruct((E, H), jnp.bfloat16),
    )(edge_attr, wet)

    mesh = plsc.VectorSubcoreMesh(core_axis_name="c", subcore_axis_name="s")
    sc_params = pltpu.CompilerParams()
    if "needs_layout_passes" in pltpu.CompilerParams.__dataclass_fields__:
        sc_params = dataclasses.replace(sc_params, needs_layout_passes=False)
    sc_a = pl.kernel(
        _sc_a_body,
        out_type=jax.ShapeDtypeStruct((2, N, H), f32),
        mesh=mesh,
        scratch_types=[
            pltpu.VMEM((CH,), jnp.int32),
            pltpu.VMEM((CH,), jnp.int32),
            pltpu.VMEM((CH,), jnp.int32),
            pltpu.VMEM((CH,), jnp.int32),
            pltpu.VMEM((CH, H), f32),
            pltpu.VMEM((CH, H), f32),
            pltpu.VMEM((CH, H // 2), jnp.int32),
            pltpu.VMEM((CH, H // 2), jnp.int32),
            pltpu.VMEM((CH, H), f32),
            pltpu.VMEM((CH, H), f32),
            pltpu.VMEM_SHARED((N, H), f32),
            pltpu.SemaphoreType.DMA((2,)),
            pltpu.SemaphoreType.DMA((2,)),
            pltpu.SemaphoreType.DMA((2,)),
            pltpu.SemaphoreType.DMA((2,)),
        ],
        compiler_params=sc_params,
    )
    v_i32 = lax.bitcast_convert_type(v.reshape(E, H // 2, 2), jnp.int32)
    s_part = sc_a(src, dst, u, v_i32)

    w2t = nd2_W.T
    ndb = nd_bias.reshape(1, H)
    wih0 = gru0_Wih.T
    whh0 = gru0_Whh.T
    bih0 = gru0_bih.reshape(1, 3 * H)
    bhh0 = gru0_bhh.reshape(1, 3 * H)
    gwt = gat_W.T
    asd_w = jnp.stack([gat_asrc, gat_adst], axis=1)
    xh, xw, asd = pl.pallas_call(
        _tc2_body,
        grid=(N // NBLK,),
        in_specs=[
            pl.BlockSpec((2, NBLK, H), lambda i: (0, i, 0)),
            pl.BlockSpec((NBLK, H), lambda i: (i, 0)),
            pl.BlockSpec((H, H), lambda i: (0, 0)),
            pl.BlockSpec((1, H), lambda i: (0, 0)),
            pl.BlockSpec((H, 3 * H), lambda i: (0, 0)),
            pl.BlockSpec((H, 3 * H), lambda i: (0, 0)),
            pl.BlockSpec((1, 3 * H), lambda i: (0, 0)),
            pl.BlockSpec((1, 3 * H), lambda i: (0, 0)),
            pl.BlockSpec((H, H), lambda i: (0, 0)),
            pl.BlockSpec((H, 2), lambda i: (0, 0)),
        ],
        out_specs=[
            pl.BlockSpec((NBLK, H), lambda i: (i, 0)),
            pl.BlockSpec((NBLK, H), lambda i: (i, 0)),
            pl.BlockSpec((NBLK, 2), lambda i: (i, 0)),
        ],
        out_shape=[jax.ShapeDtypeStruct((N, H), f32),
                   jax.ShapeDtypeStruct((N, H), f32),
                   jax.ShapeDtypeStruct((N, 2), f32)],
    )(s_part, x0, w2t, ndb, wih0, whh0, bih0, bhh0, gwt, asd_w)

    asd_flat = asd.reshape(2 * N)
    sc_b = pl.kernel(
        _sc_b_body,
        out_type=[jax.ShapeDtypeStruct((2, N, H), f32),
                  jax.ShapeDtypeStruct((2, 16, N), f32)],
        mesh=mesh,
        scratch_types=[
            pltpu.VMEM((CHB,), jnp.int32),
            pltpu.VMEM((CHB,), jnp.int32),
            pltpu.VMEM((CHB,), jnp.int32),
            pltpu.VMEM((CHB,), jnp.int32),
            pltpu.VMEM((CHB,), f32),
            pltpu.VMEM((CHB,), f32),
            pltpu.VMEM((CHB, H), f32),
            pltpu.VMEM((CHB, H), f32),
            pltpu.VMEM((2 * N,), f32),
            pltpu.VMEM((N,), f32),
            pltpu.VMEM_SHARED((N, H), f32),
            pltpu.SemaphoreType.DMA((2,)),
            pltpu.SemaphoreType.DMA((2,)),
            pltpu.SemaphoreType.DMA((2,)),
        ],
        compiler_params=sc_params,
    )
    gat_part, z_part = sc_b(src, dst, xw, asd_flat)

    gatb = gat_b.reshape(1, H)
    wih1 = gru1_Wih.T
    whh1 = gru1_Whh.T
    bih1 = gru1_bih.reshape(1, 3 * H)
    bhh1 = gru1_bhh.reshape(1, 3 * H)
    ginwt = gin_W.T
    ginb = gin_b.reshape(1, H)
    lwih = lstm_Wih.T
    lwhh = lstm_Whh.T
    lbih = lstm_bih.reshape(1, 4 * H)
    lbhh = lstm_bhh.reshape(1, 4 * H)
    l2wt = lin2_W.T
    l2b = lin2_b.reshape(1, H)
    batch3 = batch.reshape(N // NBLK, 1, NBLK)
    out = pl.pallas_call(
        _tc3_body,
        grid=(N // NBLK,),
        in_specs=[
            pl.BlockSpec((2, NBLK, H), lambda i: (0, i, 0)),
            pl.BlockSpec((1, 32, NBLK), lambda i: (i, 0, 0)),
            pl.BlockSpec((NBLK, H), lambda i: (i, 0)),
            pl.BlockSpec((1, 1, NBLK), lambda i: (i, 0, 0)),
            pl.BlockSpec((1, H), lambda i: (0, 0)),
            pl.BlockSpec((H, 3 * H), lambda i: (0, 0)),
            pl.BlockSpec((H, 3 * H), lambda i: (0, 0)),
            pl.BlockSpec((1, 3 * H), lambda i: (0, 0)),
            pl.BlockSpec((1, 3 * H), lambda i: (0, 0)),
            pl.BlockSpec((H, H), lambda i: (0, 0)),
            pl.BlockSpec((1, H), lambda i: (0, 0)),
            pl.BlockSpec((H, 4 * H), lambda i: (0, 0)),
            pl.BlockSpec((H, 4 * H), lambda i: (0, 0)),
            pl.BlockSpec((1, 4 * H), lambda i: (0, 0)),
            pl.BlockSpec((1, 4 * H), lambda i: (0, 0)),
            pl.BlockSpec((H, H), lambda i: (0, 0)),
            pl.BlockSpec((1, H), lambda i: (0, 0)),
        ],
        out_specs=pl.BlockSpec((B, H), lambda i: (0, 0)),
        out_shape=jax.ShapeDtypeStruct((B, H), f32),
        scratch_shapes=[pltpu.VMEM((B, H), f32)],
    )(gat_part, z_part.reshape(32, N // NBLK, NBLK).transpose(1, 0, 2), xh,
      batch3, gatb, wih1, whh1, bih1, bhh1, ginwt, ginb, lwih, lwhh, lbih,
      lbhh, l2wt, l2b)
    return out


# speedup vs baseline: 2.0645x; 2.0645x over previous
import dataclasses
import functools

import jax
import jax.numpy as jnp
from jax import lax
from jax.experimental import pallas as pl
from jax.experimental.pallas import tpu as pltpu
from jax.experimental.pallas import tpu_sc as plsc

N = 10000
E = 320000
H = 128
ED = 16
B = 64
T = 2

NBLK = 1000
EBLK = 4000
CH = 64
CHB = 64
NW = 32
RPT = 624


import numpy as _np

_pi = _np.arange(H)
_pg, _pj = _pi // 32, _pi % 32
QPERM = _np.where(_pj % 2 == 0, 32 * _pg + _pj // 2, 32 * _pg + 16 + _pj // 2)


def _leaky(v):
    return jnp.where(v > 0, v, 0.01 * v)


def _elu(v):
    return jnp.where(v > 0, v, jnp.exp(v) - 1.0)


def _tc1a_body(x_ref, w1t_ref, b1_ref, wxt_ref, x0_ref, u_ref):
    x0 = _leaky(jnp.dot(x_ref[...], w1t_ref[...],
                        preferred_element_type=jnp.float32) + b1_ref[...])
    x0_ref[...] = x0
    u_ref[...] = jnp.dot(x0, wxt_ref[...], preferred_element_type=jnp.float32)


def _tc1b_body(ea_ref, wet_ref, v_ref):
    v_ref[...] = jnp.dot(ea_ref[...], wet_ref[...],
                         preferred_element_type=jnp.float32)


def _tc2_body(sp_ref, x0_ref, w2t_ref, ndb_ref, wih_ref, whh_ref, bih_ref,
              bhh_ref, gwt_ref, asd_w_ref, xh_ref, xw_ref, asd_ref):
    sp = sp_ref[...]
    s = sp[0] + sp[1]
    h = _elu(jnp.dot(s, w2t_ref[...], preferred_element_type=jnp.float32)
             + ndb_ref[...])
    x0 = x0_ref[...]
    gi = jnp.dot(h, wih_ref[...], preferred_element_type=jnp.float32) + bih_ref[...]
    gh = jnp.dot(x0, whh_ref[...], preferred_element_type=jnp.float32) + bhh_ref[...]
    r = jax.nn.sigmoid(gi[:, :H] + gh[:, :H])
    z = jax.nn.sigmoid(gi[:, H:2 * H] + gh[:, H:2 * H])
    n = jnp.tanh(gi[:, 2 * H:] + r * gh[:, 2 * H:])
    xh = jax.nn.relu((1.0 - z) * n + z * x0)
    xh_ref[...] = xh
    xw = jnp.dot(xh, gwt_ref[...], preferred_element_type=jnp.float32)
    xw_ref[...] = xw
    asd_ref[...] = jnp.dot(xw, asd_w_ref[...], preferred_element_type=jnp.float32)


def _tc3_body(acc_ref, zp_ref, xh_ref, batch_ref, gatb_ref, wih_ref, whh_ref,
              bih_ref, bhh_ref, ginwt_ref, ginb_ref, lwih_ref, lwhh_ref,
              lbih_ref, lbhh_ref, l2wt_ref, l2b_ref, out_ref, pooled_ref):
    i = pl.program_id(0)
    acc = acc_ref[...]
    w = acc[0] + acc[1]
    zp = zp_ref[...][0]
    z = jnp.sum(zp, axis=0)[:, None]
    gat = w / (z + 1e-16) + gatb_ref[...]
    h2 = _elu(gat)
    xh = xh_ref[...]
    gi = jnp.dot(h2, wih_ref[...], preferred_element_type=jnp.float32) + bih_ref[...]
    gh = jnp.dot(xh, whh_ref[...], preferred_element_type=jnp.float32) + bhh_ref[...]
    r = jax.nn.sigmoid(gi[:, :H] + gh[:, :H])
    zz = jax.nn.sigmoid(gi[:, H:2 * H] + gh[:, H:2 * H])
    n = jnp.tanh(gi[:, 2 * H:] + r * gh[:, 2 * H:])
    acts = jax.nn.relu((1.0 - zz) * n + zz * xh)

    bb = batch_ref[...][0]
    seg = lax.broadcasted_iota(jnp.int32, (B, NBLK), 0)
    oh = (bb == seg).astype(jnp.float32)
    part = jnp.dot(oh, acts, preferred_element_type=jnp.float32)

    @pl.when(i == 0)
    def _():
        pooled_ref[...] = part

    @pl.when(i > 0)
    def _():
        pooled_ref[...] = pooled_ref[...] + part

    @pl.when(i == pl.num_programs(0) - 1)
    def _():
        pooled = pooled_ref[...]
        out = jax.nn.relu(pooled)
        for _ in range(T):
            ghh = _elu(jnp.dot(out + pooled, ginwt_ref[...],
                               preferred_element_type=jnp.float32) + ginb_ref[...])
            g = (jnp.dot(out, lwih_ref[...], preferred_element_type=jnp.float32)
                 + lbih_ref[...]
                 + jnp.dot(ghh, lwhh_ref[...], preferred_element_type=jnp.float32)
                 + lbhh_ref[...])
            ig = jax.nn.sigmoid(g[:, :H])
            fg = jax.nn.sigmoid(g[:, H:2 * H])
            gg = jnp.tanh(g[:, 2 * H:3 * H])
            og = jax.nn.sigmoid(g[:, 3 * H:])
            c2 = fg * ghh + ig * gg
            out = og * jnp.tanh(c2)
        out_ref[...] = (jnp.dot(out, l2wt_ref[...],
                                preferred_element_type=jnp.float32) + l2b_ref[...])


def _zero_rows(buf, nrows):
    zero16 = jnp.zeros((16,), jnp.float32)
    ncols = buf.shape[1]

    @pl.loop(0, nrows)
    def _(r):
        for j in range(ncols // 16):
            buf[r, pl.ds(j * 16, 16)] = zero16


def _zero_acc_slice(sid, zbuf, acc):
    base_r = sid * RPT
    zch = zbuf.shape[0]
    for off in range(0, RPT, zch):
        sz = min(zch, RPT - off)
        pltpu.sync_copy(zbuf.at[pl.ds(0, sz)], acc.at[pl.ds(base_r + off, sz)])

    @pl.when(sid == 15)
    def _():
        pltpu.sync_copy(zbuf.at[pl.ds(0, N - 16 * RPT)],
                        acc.at[pl.ds(16 * RPT, N - 16 * RPT)])


def _flush_acc_slice(sid, cid, acc, out_hbm):
    base_r = sid * RPT
    pltpu.sync_copy(acc.at[pl.ds(base_r, RPT)],
                    out_hbm.at[cid, pl.ds(base_r, RPT)])

    @pl.when(sid == 15)
    def _():
        pltpu.sync_copy(acc.at[pl.ds(16 * RPT, N - 16 * RPT)],
                        out_hbm.at[cid, pl.ds(16 * RPT, N - 16 * RPT)])


def _edge_pipeline(wid, nchunk, ch, src_hbm, dst_hbm, table_hbm,
                   src_idxs, dst_idxs, g_bufs, s_dst, s_g, s_sc,
                   issue_aux, wait_aux, compute, scat_bufs, acc):
    def prefetch(bn, c_next):
        ebase = c_next * ch
        pltpu.sync_copy(src_hbm.at[pl.ds(ebase, ch)], src_idxs[bn])
        pltpu.async_copy(table_hbm.at[src_idxs[bn]], g_bufs[bn], s_g.at[bn])
        pltpu.async_copy(dst_hbm.at[pl.ds(ebase, ch)], dst_idxs[bn],
                         s_dst.at[bn])
        issue_aux(bn, ebase)

    prefetch(0, wid)
    itmax = -(-nchunk // NW)
    kpair = (itmax + 2) // 2

    @pl.loop(0, kpair)
    def _(k):
        for h in (0, 1):
            it = k * 2 + h
            b, bn = h, 1 - h
            c_idx = wid + it * NW

            @pl.when((it >= 1) & (c_idx < nchunk + NW))
            def _():
                pltpu.make_async_copy(scat_bufs[bn], acc.at[dst_idxs[bn]],
                                      s_sc.at[bn]).wait()

            @pl.when(c_idx + NW < nchunk)
            def _():
                prefetch(bn, c_idx + NW)

            @pl.when(c_idx < nchunk)
            def _():
                pltpu.make_async_copy(dst_hbm.at[pl.ds(0, ch)], dst_idxs[b],
                                      s_dst.at[b]).wait()
                pltpu.make_async_copy(table_hbm.at[src_idxs[b]], g_bufs[b],
                                      s_g.at[b]).wait()
                wait_aux(b)
                compute(b)
                pltpu.async_copy(scat_bufs[b], acc.at[dst_idxs[b]],
                                 s_sc.at[b], add=True)


def _sc_a_body(src_hbm, dst_hbm, u_hbm, v_hbm, out_hbm,
               src_idx0, src_idx1, dst_idx0, dst_idx1, u0, u1, v0, v1,
               m0, m1, acc, s_dst, s_g, s_aux, s_sc):
    cid = lax.axis_index("c")
    sid = lax.axis_index("s")
    wid = sid * 2 + cid
    src_idxs, dst_idxs = (src_idx0, src_idx1), (dst_idx0, dst_idx1)
    u_bufs, v_bufs, m_bufs = (u0, u1), (v0, v1), (m0, m1)

    _zero_rows(m0, CH)
    _zero_acc_slice(sid, m0, acc)

    def issue_aux(bn, ebase):
        pltpu.async_copy(v_hbm.at[pl.ds(ebase, CH)], v_bufs[bn], s_aux.at[bn])

    def wait_aux(b):
        pltpu.make_async_copy(v_hbm.at[pl.ds(0, CH)], v_bufs[b],
                              s_aux.at[b]).wait()

    def compute(b):
        ub, vb, mb = u_bufs[b], v_bufs[b], m_bufs[b]

        @pl.loop(0, CH // 4)
        def _(q):
            r0 = q * 4
            for rr in range(4):
                r = r0 + rr
                for j in range(H // 16):
                    sl = pl.ds(j * 16, 16)
                    t = ub[r, sl] + vb[r, sl]
                    mb[r, sl] = jnp.where(t > 0, t, 0.01 * t)

    plsc.subcore_barrier()
    _edge_pipeline(wid, E // CH, CH, src_hbm, dst_hbm, u_hbm,
                   src_idxs, dst_idxs, u_bufs, s_dst, s_g, s_sc,
                   issue_aux, wait_aux, compute, m_bufs, acc)
    plsc.subcore_barrier()
    _flush_acc_slice(sid, cid, acc, out_hbm)


def _sc_b_body(src_hbm, dst_hbm, xw_hbm, asd_hbm, out_hbm, outz_hbm,
               src_idx0, src_idx1, dst_idx0, dst_idx1, pb0, pb1, xw0, xw1,
               asd_buf, z_buf, acc, s_dst, s_g, s_sc):
    cid = lax.axis_index("c")
    sid = lax.axis_index("s")
    wid = sid * 2 + cid
    lane = lax.iota(jnp.int32, 16)
    zero16 = jnp.zeros((16,), jnp.float32)
    src_idxs, dst_idxs = (src_idx0, src_idx1), (dst_idx0, dst_idx1)
    p_bufs, xw_bufs = (pb0, pb1), (xw0, xw1)

    _zero_rows(xw0, CHB)
    _zero_acc_slice(sid, xw0, acc)

    @pl.loop(0, N // 16)
    def _(r):
        z_buf[pl.ds(r * 16, 16)] = zero16

    pltpu.sync_copy(asd_hbm, asd_buf)

    def issue_aux(bn, ebase):
        pass

    def wait_aux(b):
        pass

    def compute(b):
        xb, pb_ref = xw_bufs[b], p_bufs[b]

        @pl.loop(0, CHB // 16)
        def _(j):
            sl = pl.ds(j * 16, 16)
            s16 = src_idxs[b][sl]
            d16 = dst_idxs[b][sl]
            a_s = plsc.load_gather(asd_buf, [s16 * 2])
            a_d = plsc.load_gather(asd_buf, [d16 * 2 + 1])
            e = a_s + a_d
            e = jnp.where(e > 0, e, 0.01 * e)
            p = jnp.exp(e)
            pb_ref[sl] = p
            for kk in range(16):
                plsc.addupdate_scatter(z_buf, [d16], p, mask=lane == kk)

        @pl.loop(0, CHB // 4)
        def _(q):
            r0 = q * 4
            for rr in range(4):
                r = r0 + rr
                pb = plsc.load_gather(pb_ref, [jnp.full((16,), r, jnp.int32)])
                for j in range(H // 16):
                    sl = pl.ds(j * 16, 16)
                    xb[r, sl] = xb[r, sl] * pb

    plsc.subcore_barrier()
    _edge_pipeline(wid, E // CHB, CHB, src_hbm, dst_hbm, xw_hbm,
                   src_idxs, dst_idxs, xw_bufs, s_dst, s_g, s_sc,
                   issue_aux, wait_aux, compute, xw_bufs, acc)
    plsc.subcore_barrier()
    _flush_acc_slice(sid, cid, acc, out_hbm)
    pltpu.sync_copy(z_buf, outz_hbm.at[cid, sid])


def kernel(x, edge_index, edge_attr, batch, lin1_W, lin1_b, nd1_W, nd2_W,
           nd_bias, gru0_Wih, gru0_Whh, gru0_bih, gru0_bhh, gat_W, gat_asrc,
           gat_adst, gat_b, gru1_Wih, gru1_Whh, gru1_bih, gru1_bhh, gin_W,
           gin_b, lstm_Wih, lstm_Whh, lstm_bih, lstm_bhh, lin2_W, lin2_b):
    f32 = jnp.float32
    src = edge_index[0]
    dst = edge_index[1]

    w1t = lin1_W.T
    b1 = lin1_b.reshape(1, H)
    wxt = nd1_W[:, :H].T
    wet = nd1_W[:, H:].T
    x0, u = pl.pallas_call(
        _tc1a_body,
        grid=(N // NBLK,),
        in_specs=[
            pl.BlockSpec((NBLK, H), lambda i: (i, 0)),
            pl.BlockSpec((H, H), lambda i: (0, 0)),
            pl.BlockSpec((1, H), lambda i: (0, 0)),
            pl.BlockSpec((H, H), lambda i: (0, 0)),
        ],
        out_specs=[
            pl.BlockSpec((NBLK, H), lambda i: (i, 0)),
            pl.BlockSpec((NBLK, H), lambda i: (i, 0)),
        ],
        out_shape=[jax.ShapeDtypeStruct((N, H), f32),
                   jax.ShapeDtypeStruct((N, H), f32)],
    )(x, w1t, b1, wxt)

    v = pl.pallas_call(
        _tc1b_body,
        grid=(E // EBLK,),
        in_specs=[
            pl.BlockSpec((EBLK, ED), lambda i: (i, 0)),
            pl.BlockSpec((ED, H), lambda i: (0, 0)),
        ],
        out_specs=pl.BlockSpec((EBLK, H), lambda i: (i, 0)),
        out_shape=jax.ShapeDtypeStruct((E, H), f32),
    )(edge_attr, wet)

    mesh = plsc.VectorSubcoreMesh(core_axis_name="c", subcore_axis_name="s")
    sc_params = pltpu.CompilerParams()
    if "needs_layout_passes" in pltpu.CompilerParams.__dataclass_fields__:
        sc_params = dataclasses.replace(sc_params, needs_layout_passes=False)
    sc_a = pl.kernel(
        _sc_a_body,
        out_type=jax.ShapeDtypeStruct((2, N, H), f32),
        mesh=mesh,
        scratch_types=[
            pltpu.VMEM((CH,), jnp.int32),
            pltpu.VMEM((CH,), jnp.int32),
            pltpu.VMEM((CH,), jnp.int32),
            pltpu.VMEM((CH,), jnp.int32),
            pltpu.VMEM((CH, H), f32),
            pltpu.VMEM((CH, H), f32),
            pltpu.VMEM((CH, H), f32),
            pltpu.VMEM((CH, H), f32),
            pltpu.VMEM((CH, H), f32),
            pltpu.VMEM((CH, H), f32),
            pltpu.VMEM_SHARED((N, H), f32),
            pltpu.SemaphoreType.DMA((2,)),
            pltpu.SemaphoreType.DMA((2,)),
            pltpu.SemaphoreType.DMA((2,)),
            pltpu.SemaphoreType.DMA((2,)),
        ],
        compiler_params=sc_params,
    )
    s_part = sc_a(src, dst, u, v)

    w2t = nd2_W.T
    ndb = nd_bias.reshape(1, H)
    wih0 = gru0_Wih.T
    whh0 = gru0_Whh.T
    bih0 = gru0_bih.reshape(1, 3 * H)
    bhh0 = gru0_bhh.reshape(1, 3 * H)
    gwt = gat_W.T
    asd_w = jnp.stack([gat_asrc, gat_adst], axis=1)
    xh, xw, asd = pl.pallas_call(
        _tc2_body,
        grid=(N // NBLK,),
        in_specs=[
            pl.BlockSpec((2, NBLK, H), lambda i: (0, i, 0)),
            pl.BlockSpec((NBLK, H), lambda i: (i, 0)),
            pl.BlockSpec((H, H), lambda i: (0, 0)),
            pl.BlockSpec((1, H), lambda i: (0, 0)),
            pl.BlockSpec((H, 3 * H), lambda i: (0, 0)),
            pl.BlockSpec((H, 3 * H), lambda i: (0, 0)),
            pl.BlockSpec((1, 3 * H), lambda i: (0, 0)),
            pl.BlockSpec((1, 3 * H), lambda i: (0, 0)),
            pl.BlockSpec((H, H), lambda i: (0, 0)),
            pl.BlockSpec((H, 2), lambda i: (0, 0)),
        ],
        out_specs=[
            pl.BlockSpec((NBLK, H), lambda i: (i, 0)),
            pl.BlockSpec((NBLK, H), lambda i: (i, 0)),
            pl.BlockSpec((NBLK, 2), lambda i: (i, 0)),
        ],
        out_shape=[jax.ShapeDtypeStruct((N, H), f32),
                   jax.ShapeDtypeStruct((N, H), f32),
                   jax.ShapeDtypeStruct((N, 2), f32)],
    )(s_part, x0, w2t, ndb, wih0, whh0, bih0, bhh0, gwt, asd_w)

    asd_flat = asd.reshape(2 * N)
    sc_b = pl.kernel(
        _sc_b_body,
        out_type=[jax.ShapeDtypeStruct((2, N, H), f32),
                  jax.ShapeDtypeStruct((2, 16, N), f32)],
        mesh=mesh,
        scratch_types=[
            pltpu.VMEM((CHB,), jnp.int32),
            pltpu.VMEM((CHB,), jnp.int32),
            pltpu.VMEM((CHB,), jnp.int32),
            pltpu.VMEM((CHB,), jnp.int32),
            pltpu.VMEM((CHB,), f32),
            pltpu.VMEM((CHB,), f32),
            pltpu.VMEM((CHB, H), f32),
            pltpu.VMEM((CHB, H), f32),
            pltpu.VMEM((2 * N,), f32),
            pltpu.VMEM((N,), f32),
            pltpu.VMEM_SHARED((N, H), f32),
            pltpu.SemaphoreType.DMA((2,)),
            pltpu.SemaphoreType.DMA((2,)),
            pltpu.SemaphoreType.DMA((2,)),
        ],
        compiler_params=sc_params,
    )
    gat_part, z_part = sc_b(src, dst, xw, asd_flat)

    gatb = gat_b.reshape(1, H)
    wih1 = gru1_Wih.T
    whh1 = gru1_Whh.T
    bih1 = gru1_bih.reshape(1, 3 * H)
    bhh1 = gru1_bhh.reshape(1, 3 * H)
    ginwt = gin_W.T
    ginb = gin_b.reshape(1, H)
    lwih = lstm_Wih.T
    lwhh = lstm_Whh.T
    lbih = lstm_bih.reshape(1, 4 * H)
    lbhh = lstm_bhh.reshape(1, 4 * H)
    l2wt = lin2_W.T
    l2b = lin2_b.reshape(1, H)
    batch3 = batch.reshape(N // NBLK, 1, NBLK)
    out = pl.pallas_call(
        _tc3_body,
        grid=(N // NBLK,),
        in_specs=[
            pl.BlockSpec((2, NBLK, H), lambda i: (0, i, 0)),
            pl.BlockSpec((1, 32, NBLK), lambda i: (i, 0, 0)),
            pl.BlockSpec((NBLK, H), lambda i: (i, 0)),
            pl.BlockSpec((1, 1, NBLK), lambda i: (i, 0, 0)),
            pl.BlockSpec((1, H), lambda i: (0, 0)),
            pl.BlockSpec((H, 3 * H), lambda i: (0, 0)),
            pl.BlockSpec((H, 3 * H), lambda i: (0, 0)),
            pl.BlockSpec((1, 3 * H), lambda i: (0, 0)),
            pl.BlockSpec((1, 3 * H), lambda i: (0, 0)),
            pl.BlockSpec((H, H), lambda i: (0, 0)),
            pl.BlockSpec((1, H), lambda i: (0, 0)),
            pl.BlockSpec((H, 4 * H), lambda i: (0, 0)),
            pl.BlockSpec((H, 4 * H), lambda i: (0, 0)),
            pl.BlockSpec((1, 4 * H), lambda i: (0, 0)),
            pl.BlockSpec((1, 4 * H), lambda i: (0, 0)),
            pl.BlockSpec((H, H), lambda i: (0, 0)),
            pl.BlockSpec((1, H), lambda i: (0, 0)),
        ],
        out_specs=pl.BlockSpec((B, H), lambda i: (0, 0)),
        out_shape=jax.ShapeDtypeStruct((B, H), f32),
        scratch_shapes=[pltpu.VMEM((B, H), f32)],
    )(gat_part, z_part.reshape(32, N // NBLK, NBLK).transpose(1, 0, 2), xh,
      batch3, gatb, wih1, whh1, bih1, bhh1, ginwt, ginb, lwih, lwhh, lbih,
      lbhh, l2wt, l2b)
    return out


# speedup vs baseline: 2.2903x; 1.1094x over previous
import dataclasses
import functools

import jax
import jax.numpy as jnp
from jax import lax
from jax.experimental import pallas as pl
from jax.experimental.pallas import tpu as pltpu
from jax.experimental.pallas import tpu_sc as plsc

N = 10000
E = 320000
H = 128
ED = 16
B = 64
T = 2

NBLK = 1000
EBLK = 4000
CH = 64
CHB = 64
NW = 32
RPT = 624


import numpy as _np

_pi = _np.arange(H)
_pg, _pj = _pi // 32, _pi % 32
QPERM = _np.where(_pj % 2 == 0, 32 * _pg + _pj // 2, 32 * _pg + 16 + _pj // 2)


def _leaky(v):
    return jnp.where(v > 0, v, 0.01 * v)


def _elu(v):
    return jnp.where(v > 0, v, jnp.exp(v) - 1.0)


def _tc1a_body(x_ref, w1t_ref, b1_ref, wxt_ref, x0_ref, u_ref):
    x0 = _leaky(jnp.dot(x_ref[...], w1t_ref[...],
                        preferred_element_type=jnp.float32) + b1_ref[...])
    x0_ref[...] = x0
    u_ref[...] = jnp.dot(x0, wxt_ref[...], preferred_element_type=jnp.float32)


def _tc1b_body(ea_ref, wet_ref, v_ref):
    v_ref[...] = jnp.dot(ea_ref[...], wet_ref[...],
                         preferred_element_type=jnp.float32)


def _tc2_body(sp_ref, x0_ref, w2t_ref, ndb_ref, wih_ref, whh_ref, bih_ref,
              bhh_ref, gwt_ref, asd_w_ref, xh_ref, xw_ref, asd_ref):
    sp = sp_ref[...]
    s = sp[0] + sp[1]
    h = _elu(jnp.dot(s, w2t_ref[...], preferred_element_type=jnp.float32)
             + ndb_ref[...])
    x0 = x0_ref[...]
    gi = jnp.dot(h, wih_ref[...], preferred_element_type=jnp.float32) + bih_ref[...]
    gh = jnp.dot(x0, whh_ref[...], preferred_element_type=jnp.float32) + bhh_ref[...]
    r = jax.nn.sigmoid(gi[:, :H] + gh[:, :H])
    z = jax.nn.sigmoid(gi[:, H:2 * H] + gh[:, H:2 * H])
    n = jnp.tanh(gi[:, 2 * H:] + r * gh[:, 2 * H:])
    xh = jax.nn.relu((1.0 - z) * n + z * x0)
    xh_ref[...] = xh
    xw = jnp.dot(xh, gwt_ref[...], preferred_element_type=jnp.float32)
    xw_ref[...] = xw
    asd_ref[...] = jnp.dot(xw, asd_w_ref[...], preferred_element_type=jnp.float32)


def _tc3_body(acc_ref, zp_ref, xh_ref, batch_ref, gatb_ref, wih_ref, whh_ref,
              bih_ref, bhh_ref, ginwt_ref, ginb_ref, lwih_ref, lwhh_ref,
              lbih_ref, lbhh_ref, l2wt_ref, l2b_ref, out_ref, pooled_ref):
    i = pl.program_id(0)
    acc = acc_ref[...]
    w = acc[0] + acc[1]
    zp = zp_ref[...][0]
    z = jnp.sum(zp, axis=0)[:, None]
    gat = w / (z + 1e-16) + gatb_ref[...]
    h2 = _elu(gat)
    xh = xh_ref[...]
    gi = jnp.dot(h2, wih_ref[...], preferred_element_type=jnp.float32) + bih_ref[...]
    gh = jnp.dot(xh, whh_ref[...], preferred_element_type=jnp.float32) + bhh_ref[...]
    r = jax.nn.sigmoid(gi[:, :H] + gh[:, :H])
    zz = jax.nn.sigmoid(gi[:, H:2 * H] + gh[:, H:2 * H])
    n = jnp.tanh(gi[:, 2 * H:] + r * gh[:, 2 * H:])
    acts = jax.nn.relu((1.0 - zz) * n + zz * xh)

    bb = batch_ref[...][0]
    seg = lax.broadcasted_iota(jnp.int32, (B, NBLK), 0)
    oh = (bb == seg).astype(jnp.float32)
    part = jnp.dot(oh, acts, preferred_element_type=jnp.float32)

    @pl.when(i == 0)
    def _():
        pooled_ref[...] = part

    @pl.when(i > 0)
    def _():
        pooled_ref[...] = pooled_ref[...] + part

    @pl.when(i == pl.num_programs(0) - 1)
    def _():
        pooled = pooled_ref[...]
        out = jax.nn.relu(pooled)
        for _ in range(T):
            ghh = _elu(jnp.dot(out + pooled, ginwt_ref[...],
                               preferred_element_type=jnp.float32) + ginb_ref[...])
            g = (jnp.dot(out, lwih_ref[...], preferred_element_type=jnp.float32)
                 + lbih_ref[...]
                 + jnp.dot(ghh, lwhh_ref[...], preferred_element_type=jnp.float32)
                 + lbhh_ref[...])
            ig = jax.nn.sigmoid(g[:, :H])
            fg = jax.nn.sigmoid(g[:, H:2 * H])
            gg = jnp.tanh(g[:, 2 * H:3 * H])
            og = jax.nn.sigmoid(g[:, 3 * H:])
            c2 = fg * ghh + ig * gg
            out = og * jnp.tanh(c2)
        out_ref[...] = (jnp.dot(out, l2wt_ref[...],
                                preferred_element_type=jnp.float32) + l2b_ref[...])


def _zero_rows(buf, nrows):
    zero16 = jnp.zeros((16,), jnp.float32)
    ncols = buf.shape[1]

    @pl.loop(0, nrows)
    def _(r):
        for j in range(ncols // 16):
            buf[r, pl.ds(j * 16, 16)] = zero16


def _zero_acc_slice(sid, zbuf, acc):
    base_r = sid * RPT
    zch = zbuf.shape[0]
    for off in range(0, RPT, zch):
        sz = min(zch, RPT - off)
        pltpu.sync_copy(zbuf.at[pl.ds(0, sz)], acc.at[pl.ds(base_r + off, sz)])

    @pl.when(sid == 15)
    def _():
        pltpu.sync_copy(zbuf.at[pl.ds(0, N - 16 * RPT)],
                        acc.at[pl.ds(16 * RPT, N - 16 * RPT)])


def _flush_acc_slice(sid, cid, acc, out_hbm):
    base_r = sid * RPT
    pltpu.sync_copy(acc.at[pl.ds(base_r, RPT)],
                    out_hbm.at[cid, pl.ds(base_r, RPT)])

    @pl.when(sid == 15)
    def _():
        pltpu.sync_copy(acc.at[pl.ds(16 * RPT, N - 16 * RPT)],
                        out_hbm.at[cid, pl.ds(16 * RPT, N - 16 * RPT)])


def _edge_pipeline(wid, nchunk, ch, src_hbm, dst_hbm, table_hbm,
                   src_idxs, dst_idxs, g_bufs, s_src, s_dst, s_g, s_sc,
                   issue_aux, wait_aux, compute, scat_bufs, acc):
    def issue_front(c_idx, gb, d3):
        ebase = c_idx * ch
        pltpu.async_copy(table_hbm.at[src_idxs[gb]], g_bufs[gb], s_g.at[gb])
        pltpu.async_copy(dst_hbm.at[pl.ds(ebase, ch)], dst_idxs[d3],
                         s_dst.at[d3])
        issue_aux(gb, ebase)

    pltpu.sync_copy(src_hbm.at[pl.ds(wid * ch, ch)], src_idxs[0])
    issue_front(wid, 0, 0)
    pltpu.async_copy(src_hbm.at[pl.ds((wid + NW) * ch, ch)], src_idxs[1],
                     s_src.at[1])

    itmax = -(-nchunk // NW)
    k6 = (itmax + 6) // 6

    @pl.loop(0, k6)
    def _(k):
        for h in range(6):
            it = k * 6 + h
            b, bn = h % 2, 1 - h % 2
            d3, d3n, d3p = h % 3, (h + 1) % 3, (h + 2) % 3
            c_idx = wid + it * NW

            @pl.when(c_idx + NW < nchunk)
            def _():
                pltpu.make_async_copy(src_hbm.at[pl.ds(0, ch)], src_idxs[bn],
                                      s_src.at[bn]).wait()
                issue_front(c_idx + NW, bn, d3n)

            @pl.when(c_idx < nchunk)
            def _():
                pltpu.make_async_copy(dst_hbm.at[pl.ds(0, ch)], dst_idxs[d3],
                                      s_dst.at[d3]).wait()
                pltpu.make_async_copy(table_hbm.at[src_idxs[b]], g_bufs[b],
                                      s_g.at[b]).wait()
                wait_aux(b)
                compute(b, d3)

                @pl.when(it >= 1)
                def _():
                    pltpu.make_async_copy(scat_bufs[bn], acc.at[dst_idxs[d3p]],
                                          s_sc.at[bn]).wait()

                pltpu.async_copy(scat_bufs[b], acc.at[dst_idxs[d3]],
                                 s_sc.at[b], add=True)

            @pl.when((c_idx >= nchunk) & (c_idx < nchunk + NW) & (it >= 1))
            def _():
                pltpu.make_async_copy(scat_bufs[bn], acc.at[dst_idxs[d3p]],
                                      s_sc.at[bn]).wait()

            @pl.when(c_idx + 2 * NW < nchunk)
            def _():
                pltpu.async_copy(src_hbm.at[pl.ds((c_idx + 2 * NW) * ch, ch)],
                                 src_idxs[b], s_src.at[b])


def _sc_a_body(src_hbm, dst_hbm, u_hbm, v_hbm, out_hbm,
               src_idx0, src_idx1, dst_idx0, dst_idx1, dst_idx2, u0, u1,
               v0, v1, m0, m1, acc, s_src, s_dst, s_g, s_aux, s_sc):
    cid = lax.axis_index("c")
    sid = lax.axis_index("s")
    wid = sid * 2 + cid
    src_idxs = (src_idx0, src_idx1)
    dst_idxs = (dst_idx0, dst_idx1, dst_idx2)
    u_bufs, v_bufs, m_bufs = (u0, u1), (v0, v1), (m0, m1)

    _zero_rows(m0, CH)
    _zero_acc_slice(sid, m0, acc)

    def issue_aux(bn, ebase):
        pltpu.async_copy(v_hbm.at[pl.ds(ebase, CH)], v_bufs[bn], s_aux.at[bn])

    def wait_aux(b):
        pltpu.make_async_copy(v_hbm.at[pl.ds(0, CH)], v_bufs[b],
                              s_aux.at[b]).wait()

    def compute(b, d3):
        ub, vb, mb = u_bufs[b], v_bufs[b], m_bufs[b]

        @pl.loop(0, CH // 4)
        def _(q):
            r0 = q * 4
            for rr in range(4):
                r = r0 + rr
                for j in range(H // 16):
                    sl = pl.ds(j * 16, 16)
                    t = ub[r, sl] + vb[r, sl]
                    mb[r, sl] = jnp.where(t > 0, t, 0.01 * t)

    plsc.subcore_barrier()
    _edge_pipeline(wid, E // CH, CH, src_hbm, dst_hbm, u_hbm,
                   src_idxs, dst_idxs, u_bufs, s_src, s_dst, s_g, s_sc,
                   issue_aux, wait_aux, compute, m_bufs, acc)
    plsc.subcore_barrier()
    _flush_acc_slice(sid, cid, acc, out_hbm)


def _sc_b_body(src_hbm, dst_hbm, xw_hbm, asd_hbm, out_hbm, outz_hbm,
               src_idx0, src_idx1, dst_idx0, dst_idx1, dst_idx2, pb0, pb1,
               xw0, xw1, asd_buf, z_buf, acc, s_src, s_dst, s_g, s_sc):
    cid = lax.axis_index("c")
    sid = lax.axis_index("s")
    wid = sid * 2 + cid
    lane = lax.iota(jnp.int32, 16)
    zero16 = jnp.zeros((16,), jnp.float32)
    src_idxs = (src_idx0, src_idx1)
    dst_idxs = (dst_idx0, dst_idx1, dst_idx2)
    p_bufs, xw_bufs = (pb0, pb1), (xw0, xw1)

    _zero_rows(xw0, CHB)
    _zero_acc_slice(sid, xw0, acc)

    @pl.loop(0, N // 16)
    def _(r):
        z_buf[pl.ds(r * 16, 16)] = zero16

    pltpu.sync_copy(asd_hbm, asd_buf)

    def issue_aux(bn, ebase):
        pass

    def wait_aux(b):
        pass

    def compute(b, d3):
        xb, pb_ref = xw_bufs[b], p_bufs[b]

        @pl.loop(0, CHB // 16)
        def _(j):
            sl = pl.ds(j * 16, 16)
            s16 = src_idxs[b][sl]
            d16 = dst_idxs[d3][sl]
            a_s = plsc.load_gather(asd_buf, [s16 * 2])
            a_d = plsc.load_gather(asd_buf, [d16 * 2 + 1])
            e = a_s + a_d
            e = jnp.where(e > 0, e, 0.01 * e)
            p = jnp.exp(e)
            pb_ref[sl] = p
            for kk in range(16):
                plsc.addupdate_scatter(z_buf, [d16], p, mask=lane == kk)

        @pl.loop(0, CHB // 4)
        def _(q):
            r0 = q * 4
            for rr in range(4):
                r = r0 + rr
                pb = plsc.load_gather(pb_ref, [jnp.full((16,), r, jnp.int32)])
                for j in range(H // 16):
                    sl = pl.ds(j * 16, 16)
                    xb[r, sl] = xb[r, sl] * pb

    plsc.subcore_barrier()
    _edge_pipeline(wid, E // CHB, CHB, src_hbm, dst_hbm, xw_hbm,
                   src_idxs, dst_idxs, xw_bufs, s_src, s_dst, s_g, s_sc,
                   issue_aux, wait_aux, compute, xw_bufs, acc)
    plsc.subcore_barrier()
    _flush_acc_slice(sid, cid, acc, out_hbm)
    pltpu.sync_copy(z_buf, outz_hbm.at[cid, sid])


def kernel(x, edge_index, edge_attr, batch, lin1_W, lin1_b, nd1_W, nd2_W,
           nd_bias, gru0_Wih, gru0_Whh, gru0_bih, gru0_bhh, gat_W, gat_asrc,
           gat_adst, gat_b, gru1_Wih, gru1_Whh, gru1_bih, gru1_bhh, gin_W,
           gin_b, lstm_Wih, lstm_Whh, lstm_bih, lstm_bhh, lin2_W, lin2_b):
    f32 = jnp.float32
    src = edge_index[0]
    dst = edge_index[1]

    w1t = lin1_W.T
    b1 = lin1_b.reshape(1, H)
    wxt = nd1_W[:, :H].T
    wet = nd1_W[:, H:].T
    x0, u = pl.pallas_call(
        _tc1a_body,
        grid=(N // NBLK,),
        in_specs=[
            pl.BlockSpec((NBLK, H), lambda i: (i, 0)),
            pl.BlockSpec((H, H), lambda i: (0, 0)),
            pl.BlockSpec((1, H), lambda i: (0, 0)),
            pl.BlockSpec((H, H), lambda i: (0, 0)),
        ],
        out_specs=[
            pl.BlockSpec((NBLK, H), lambda i: (i, 0)),
            pl.BlockSpec((NBLK, H), lambda i: (i, 0)),
        ],
        out_shape=[jax.ShapeDtypeStruct((N, H), f32),
                   jax.ShapeDtypeStruct((N, H), f32)],
    )(x, w1t, b1, wxt)

    v = pl.pallas_call(
        _tc1b_body,
        grid=(E // EBLK,),
        in_specs=[
            pl.BlockSpec((EBLK, ED), lambda i: (i, 0)),
            pl.BlockSpec((ED, H), lambda i: (0, 0)),
        ],
        out_specs=pl.BlockSpec((EBLK, H), lambda i: (i, 0)),
        out_shape=jax.ShapeDtypeStruct((E, H), f32),
    )(edge_attr, wet)

    mesh = plsc.VectorSubcoreMesh(core_axis_name="c", subcore_axis_name="s")
    sc_params = pltpu.CompilerParams()
    if "needs_layout_passes" in pltpu.CompilerParams.__dataclass_fields__:
        sc_params = dataclasses.replace(sc_params, needs_layout_passes=False)
    sc_a = pl.kernel(
        _sc_a_body,
        out_type=jax.ShapeDtypeStruct((2, N, H), f32),
        mesh=mesh,
        scratch_types=[
            pltpu.VMEM((CH,), jnp.int32),
            pltpu.VMEM((CH,), jnp.int32),
            pltpu.VMEM((CH,), jnp.int32),
            pltpu.VMEM((CH,), jnp.int32),
            pltpu.VMEM((CH,), jnp.int32),
            pltpu.VMEM((CH, H), f32),
            pltpu.VMEM((CH, H), f32),
            pltpu.VMEM((CH, H), f32),
            pltpu.VMEM((CH, H), f32),
            pltpu.VMEM((CH, H), f32),
            pltpu.VMEM((CH, H), f32),
            pltpu.VMEM_SHARED((N, H), f32),
            pltpu.SemaphoreType.DMA((2,)),
            pltpu.SemaphoreType.DMA((3,)),
            pltpu.SemaphoreType.DMA((2,)),
            pltpu.SemaphoreType.DMA((2,)),
            pltpu.SemaphoreType.DMA((2,)),
        ],
        compiler_params=sc_params,
    )
    s_part = sc_a(src, dst, u, v)

    w2t = nd2_W.T
    ndb = nd_bias.reshape(1, H)
    wih0 = gru0_Wih.T
    whh0 = gru0_Whh.T
    bih0 = gru0_bih.reshape(1, 3 * H)
    bhh0 = gru0_bhh.reshape(1, 3 * H)
    gwt = gat_W.T
    asd_w = jnp.stack([gat_asrc, gat_adst], axis=1)
    xh, xw, asd = pl.pallas_call(
        _tc2_body,
        grid=(N // NBLK,),
        in_specs=[
            pl.BlockSpec((2, NBLK, H), lambda i: (0, i, 0)),
            pl.BlockSpec((NBLK, H), lambda i: (i, 0)),
            pl.BlockSpec((H, H), lambda i: (0, 0)),
            pl.BlockSpec((1, H), lambda i: (0, 0)),
            pl.BlockSpec((H, 3 * H), lambda i: (0, 0)),
            pl.BlockSpec((H, 3 * H), lambda i: (0, 0)),
            pl.BlockSpec((1, 3 * H), lambda i: (0, 0)),
            pl.BlockSpec((1, 3 * H), lambda i: (0, 0)),
            pl.BlockSpec((H, H), lambda i: (0, 0)),
            pl.BlockSpec((H, 2), lambda i: (0, 0)),
        ],
        out_specs=[
            pl.BlockSpec((NBLK, H), lambda i: (i, 0)),
            pl.BlockSpec((NBLK, H), lambda i: (i, 0)),
            pl.BlockSpec((NBLK, 2), lambda i: (i, 0)),
        ],
        out_shape=[jax.ShapeDtypeStruct((N, H), f32),
                   jax.ShapeDtypeStruct((N, H), f32),
                   jax.ShapeDtypeStruct((N, 2), f32)],
    )(s_part, x0, w2t, ndb, wih0, whh0, bih0, bhh0, gwt, asd_w)

    asd_flat = asd.reshape(2 * N)
    sc_b = pl.kernel(
        _sc_b_body,
        out_type=[jax.ShapeDtypeStruct((2, N, H), f32),
                  jax.ShapeDtypeStruct((2, 16, N), f32)],
        mesh=mesh,
        scratch_types=[
            pltpu.VMEM((CHB,), jnp.int32),
            pltpu.VMEM((CHB,), jnp.int32),
            pltpu.VMEM((CHB,), jnp.int32),
            pltpu.VMEM((CHB,), jnp.int32),
            pltpu.VMEM((CHB,), jnp.int32),
            pltpu.VMEM((CHB,), f32),
            pltpu.VMEM((CHB,), f32),
            pltpu.VMEM((CHB, H), f32),
            pltpu.VMEM((CHB, H), f32),
            pltpu.VMEM((2 * N,), f32),
            pltpu.VMEM((N,), f32),
            pltpu.VMEM_SHARED((N, H), f32),
            pltpu.SemaphoreType.DMA((2,)),
            pltpu.SemaphoreType.DMA((3,)),
            pltpu.SemaphoreType.DMA((2,)),
            pltpu.SemaphoreType.DMA((2,)),
        ],
        compiler_params=sc_params,
    )
    gat_part, z_part = sc_b(src, dst, xw, asd_flat)

    gatb = gat_b.reshape(1, H)
    wih1 = gru1_Wih.T
    whh1 = gru1_Whh.T
    bih1 = gru1_bih.reshape(1, 3 * H)
    bhh1 = gru1_bhh.reshape(1, 3 * H)
    ginwt = gin_W.T
    ginb = gin_b.reshape(1, H)
    lwih = lstm_Wih.T
    lwhh = lstm_Whh.T
    lbih = lstm_bih.reshape(1, 4 * H)
    lbhh = lstm_bhh.reshape(1, 4 * H)
    l2wt = lin2_W.T
    l2b = lin2_b.reshape(1, H)
    batch3 = batch.reshape(N // NBLK, 1, NBLK)
    out = pl.pallas_call(
        _tc3_body,
        grid=(N // NBLK,),
        in_specs=[
            pl.BlockSpec((2, NBLK, H), lambda i: (0, i, 0)),
            pl.BlockSpec((1, 32, NBLK), lambda i: (i, 0, 0)),
            pl.BlockSpec((NBLK, H), lambda i: (i, 0)),
            pl.BlockSpec((1, 1, NBLK), lambda i: (i, 0, 0)),
            pl.BlockSpec((1, H), lambda i: (0, 0)),
            pl.BlockSpec((H, 3 * H), lambda i: (0, 0)),
            pl.BlockSpec((H, 3 * H), lambda i: (0, 0)),
            pl.BlockSpec((1, 3 * H), lambda i: (0, 0)),
            pl.BlockSpec((1, 3 * H), lambda i: (0, 0)),
            pl.BlockSpec((H, H), lambda i: (0, 0)),
            pl.BlockSpec((1, H), lambda i: (0, 0)),
            pl.BlockSpec((H, 4 * H), lambda i: (0, 0)),
            pl.BlockSpec((H, 4 * H), lambda i: (0, 0)),
            pl.BlockSpec((1, 4 * H), lambda i: (0, 0)),
            pl.BlockSpec((1, 4 * H), lambda i: (0, 0)),
            pl.BlockSpec((H, H), lambda i: (0, 0)),
            pl.BlockSpec((1, H), lambda i: (0, 0)),
        ],
        out_specs=pl.BlockSpec((B, H), lambda i: (0, 0)),
        out_shape=jax.ShapeDtypeStruct((B, H), f32),
        scratch_shapes=[pltpu.VMEM((B, H), f32)],
    )(gat_part, z_part.reshape(32, N // NBLK, NBLK).transpose(1, 0, 2), xh,
      batch3, gatb, wih1, whh1, bih1, bhh1, ginwt, ginb, lwih, lwhh, lbih,
      lbhh, l2wt, l2b)
    return out


# speedup vs baseline: 2.3442x; 1.0235x over previous
import dataclasses
import functools

import jax
import jax.numpy as jnp
from jax import lax
from jax.experimental import pallas as pl
from jax.experimental.pallas import tpu as pltpu
from jax.experimental.pallas import tpu_sc as plsc

N = 10000
E = 320000
H = 128
ED = 16
B = 64
T = 2

NBLK = 1000
EBLK = 16000
CH = 64
CHB = 64
NW = 32
RPT = 624


import numpy as _np

_pi = _np.arange(H)
_pg, _pj = _pi // 32, _pi % 32
QPERM = _np.where(_pj % 2 == 0, 32 * _pg + _pj // 2, 32 * _pg + 16 + _pj // 2)


def _leaky(v):
    return jnp.where(v > 0, v, 0.01 * v)


def _elu(v):
    return jnp.where(v > 0, v, jnp.exp(v) - 1.0)


def _tc1a_body(x_ref, w1t_ref, b1_ref, wxt_ref, x0_ref, u_ref):
    x0 = _leaky(jnp.dot(x_ref[...], w1t_ref[...],
                        preferred_element_type=jnp.float32) + b1_ref[...])
    x0_ref[...] = x0
    u_ref[...] = jnp.dot(x0, wxt_ref[...], preferred_element_type=jnp.float32)


def _tc1b_body(ea_ref, wet_ref, v_ref):
    v_ref[...] = jnp.dot(ea_ref[...], wet_ref[...],
                         preferred_element_type=jnp.float32)


def _tc2_body(sp_ref, x0_ref, w2t_ref, ndb_ref, wih_ref, whh_ref, bih_ref,
              bhh_ref, gwt_ref, asd_w_ref, xh_ref, xw_ref, asd_ref):
    sp = sp_ref[...]
    s = sp[0] + sp[1]
    h = _elu(jnp.dot(s, w2t_ref[...], preferred_element_type=jnp.float32)
             + ndb_ref[...])
    x0 = x0_ref[...]
    gi = jnp.dot(h, wih_ref[...], preferred_element_type=jnp.float32) + bih_ref[...]
    gh = jnp.dot(x0, whh_ref[...], preferred_element_type=jnp.float32) + bhh_ref[...]
    r = jax.nn.sigmoid(gi[:, :H] + gh[:, :H])
    z = jax.nn.sigmoid(gi[:, H:2 * H] + gh[:, H:2 * H])
    n = jnp.tanh(gi[:, 2 * H:] + r * gh[:, 2 * H:])
    xh = jax.nn.relu((1.0 - z) * n + z * x0)
    xh_ref[...] = xh
    xw = jnp.dot(xh, gwt_ref[...], preferred_element_type=jnp.float32)
    xw_ref[...] = xw
    asd_ref[...] = jnp.dot(xw, asd_w_ref[...], preferred_element_type=jnp.float32)


def _tc3_body(acc_ref, zp_ref, xh_ref, batch_ref, gatb_ref, wih_ref, whh_ref,
              bih_ref, bhh_ref, ginwt_ref, ginb_ref, lwih_ref, lwhh_ref,
              lbih_ref, lbhh_ref, l2wt_ref, l2b_ref, out_ref, pooled_ref):
    i = pl.program_id(0)
    acc = acc_ref[...]
    w = acc[0] + acc[1]
    zp = zp_ref[...][0]
    z = jnp.sum(zp, axis=0)[:, None]
    gat = w / (z + 1e-16) + gatb_ref[...]
    h2 = _elu(gat)
    xh = xh_ref[...]
    gi = jnp.dot(h2, wih_ref[...], preferred_element_type=jnp.float32) + bih_ref[...]
    gh = jnp.dot(xh, whh_ref[...], preferred_element_type=jnp.float32) + bhh_ref[...]
    r = jax.nn.sigmoid(gi[:, :H] + gh[:, :H])
    zz = jax.nn.sigmoid(gi[:, H:2 * H] + gh[:, H:2 * H])
    n = jnp.tanh(gi[:, 2 * H:] + r * gh[:, 2 * H:])
    acts = jax.nn.relu((1.0 - zz) * n + zz * xh)

    bb = batch_ref[...][0]
    seg = lax.broadcasted_iota(jnp.int32, (B, NBLK), 0)
    oh = (bb == seg).astype(jnp.float32)
    part = jnp.dot(oh, acts, preferred_element_type=jnp.float32)

    @pl.when(i == 0)
    def _():
        pooled_ref[...] = part

    @pl.when(i > 0)
    def _():
        pooled_ref[...] = pooled_ref[...] + part

    @pl.when(i == pl.num_programs(0) - 1)
    def _():
        pooled = pooled_ref[...]
        out = jax.nn.relu(pooled)
        for _ in range(T):
            ghh = _elu(jnp.dot(out + pooled, ginwt_ref[...],
                               preferred_element_type=jnp.float32) + ginb_ref[...])
            g = (jnp.dot(out, lwih_ref[...], preferred_element_type=jnp.float32)
                 + lbih_ref[...]
                 + jnp.dot(ghh, lwhh_ref[...], preferred_element_type=jnp.float32)
                 + lbhh_ref[...])
            ig = jax.nn.sigmoid(g[:, :H])
            fg = jax.nn.sigmoid(g[:, H:2 * H])
            gg = jnp.tanh(g[:, 2 * H:3 * H])
            og = jax.nn.sigmoid(g[:, 3 * H:])
            c2 = fg * ghh + ig * gg
            out = og * jnp.tanh(c2)
        out_ref[...] = (jnp.dot(out, l2wt_ref[...],
                                preferred_element_type=jnp.float32) + l2b_ref[...])


def _zero_rows(buf, nrows):
    zero16 = jnp.zeros((16,), jnp.float32)
    ncols = buf.shape[1]

    @pl.loop(0, nrows)
    def _(r):
        for j in range(ncols // 16):
            buf[r, pl.ds(j * 16, 16)] = zero16


def _zero_acc_slice(sid, zbuf, acc):
    base_r = sid * RPT
    zch = zbuf.shape[0]
    for off in range(0, RPT, zch):
        sz = min(zch, RPT - off)
        pltpu.sync_copy(zbuf.at[pl.ds(0, sz)], acc.at[pl.ds(base_r + off, sz)])

    @pl.when(sid == 15)
    def _():
        pltpu.sync_copy(zbuf.at[pl.ds(0, N - 16 * RPT)],
                        acc.at[pl.ds(16 * RPT, N - 16 * RPT)])


def _flush_acc_slice(sid, cid, acc, out_hbm):
    base_r = sid * RPT
    pltpu.sync_copy(acc.at[pl.ds(base_r, RPT)],
                    out_hbm.at[cid, pl.ds(base_r, RPT)])

    @pl.when(sid == 15)
    def _():
        pltpu.sync_copy(acc.at[pl.ds(16 * RPT, N - 16 * RPT)],
                        out_hbm.at[cid, pl.ds(16 * RPT, N - 16 * RPT)])


def _edge_pipeline(wid, nchunk, ch, src_hbm, dst_hbm, table_hbm,
                   src_idxs, dst_idxs, g_bufs, s_src, s_dst, s_g, s_sc,
                   issue_aux, wait_aux, compute, scat_bufs, acc):
    def issue_front(c_idx, gb, d3):
        ebase = c_idx * ch
        pltpu.async_copy(table_hbm.at[src_idxs[gb]], g_bufs[gb], s_g.at[gb])
        pltpu.async_copy(dst_hbm.at[pl.ds(ebase, ch)], dst_idxs[d3],
                         s_dst.at[d3])
        issue_aux(gb, ebase)

    pltpu.sync_copy(src_hbm.at[pl.ds(wid * ch, ch)], src_idxs[0])
    issue_front(wid, 0, 0)
    pltpu.async_copy(src_hbm.at[pl.ds((wid + NW) * ch, ch)], src_idxs[1],
                     s_src.at[1])

    itmax = -(-nchunk // NW)
    k6 = (itmax + 6) // 6

    @pl.loop(0, k6)
    def _(k):
        for h in range(6):
            it = k * 6 + h
            b, bn = h % 2, 1 - h % 2
            d3, d3n, d3p = h % 3, (h + 1) % 3, (h + 2) % 3
            c_idx = wid + it * NW

            @pl.when(c_idx + NW < nchunk)
            def _():
                pltpu.make_async_copy(src_hbm.at[pl.ds(0, ch)], src_idxs[bn],
                                      s_src.at[bn]).wait()
                issue_front(c_idx + NW, bn, d3n)

            @pl.when(c_idx < nchunk)
            def _():
                pltpu.make_async_copy(dst_hbm.at[pl.ds(0, ch)], dst_idxs[d3],
                                      s_dst.at[d3]).wait()
                pltpu.make_async_copy(table_hbm.at[src_idxs[b]], g_bufs[b],
                                      s_g.at[b]).wait()
                wait_aux(b)
                compute(b, d3)

                @pl.when(it >= 1)
                def _():
                    pltpu.make_async_copy(scat_bufs[bn], acc.at[dst_idxs[d3p]],
                                          s_sc.at[bn]).wait()

                pltpu.async_copy(scat_bufs[b], acc.at[dst_idxs[d3]],
                                 s_sc.at[b], add=True)

            @pl.when((c_idx >= nchunk) & (c_idx < nchunk + NW) & (it >= 1))
            def _():
                pltpu.make_async_copy(scat_bufs[bn], acc.at[dst_idxs[d3p]],
                                      s_sc.at[bn]).wait()

            @pl.when(c_idx + 2 * NW < nchunk)
            def _():
                pltpu.async_copy(src_hbm.at[pl.ds((c_idx + 2 * NW) * ch, ch)],
                                 src_idxs[b], s_src.at[b])


def _sc_a_body(src_hbm, dst_hbm, u_hbm, v_hbm, out_hbm,
               src_idx0, src_idx1, dst_idx0, dst_idx1, dst_idx2, u0, u1,
               v0, v1, m0, m1, acc, s_src, s_dst, s_g, s_aux, s_sc):
    cid = lax.axis_index("c")
    sid = lax.axis_index("s")
    wid = sid * 2 + cid
    src_idxs = (src_idx0, src_idx1)
    dst_idxs = (dst_idx0, dst_idx1, dst_idx2)
    u_bufs, v_bufs, m_bufs = (u0, u1), (v0, v1), (m0, m1)

    _zero_rows(m0, CH)
    _zero_acc_slice(sid, m0, acc)

    def issue_aux(bn, ebase):
        pltpu.async_copy(v_hbm.at[pl.ds(ebase, CH)], v_bufs[bn], s_aux.at[bn])

    def wait_aux(b):
        pltpu.make_async_copy(v_hbm.at[pl.ds(0, CH)], v_bufs[b],
                              s_aux.at[b]).wait()

    def compute(b, d3):
        ub, vb, mb = u_bufs[b], v_bufs[b], m_bufs[b]

        @pl.loop(0, CH // 4)
        def _(q):
            r0 = q * 4
            for rr in range(4):
                r = r0 + rr
                for j in range(H // 16):
                    sl = pl.ds(j * 16, 16)
                    t = ub[r, sl] + vb[r, sl]
                    mb[r, sl] = jnp.where(t > 0, t, 0.01 * t)

    plsc.subcore_barrier()
    _edge_pipeline(wid, E // CH, CH, src_hbm, dst_hbm, u_hbm,
                   src_idxs, dst_idxs, u_bufs, s_src, s_dst, s_g, s_sc,
                   issue_aux, wait_aux, compute, m_bufs, acc)
    plsc.subcore_barrier()
    _flush_acc_slice(sid, cid, acc, out_hbm)


def _sc_b_body(src_hbm, dst_hbm, xw_hbm, asd_hbm, out_hbm, outz_hbm,
               src_idx0, src_idx1, dst_idx0, dst_idx1, dst_idx2, pb0, pb1,
               xw0, xw1, asd_buf, z_buf, acc, s_src, s_dst, s_g, s_sc):
    cid = lax.axis_index("c")
    sid = lax.axis_index("s")
    wid = sid * 2 + cid
    lane = lax.iota(jnp.int32, 16)
    zero16 = jnp.zeros((16,), jnp.float32)
    src_idxs = (src_idx0, src_idx1)
    dst_idxs = (dst_idx0, dst_idx1, dst_idx2)
    p_bufs, xw_bufs = (pb0, pb1), (xw0, xw1)

    _zero_rows(xw0, CHB)
    _zero_acc_slice(sid, xw0, acc)

    @pl.loop(0, N // 16)
    def _(r):
        z_buf[pl.ds(r * 16, 16)] = zero16

    pltpu.sync_copy(asd_hbm, asd_buf)

    def issue_aux(bn, ebase):
        pass

    def wait_aux(b):
        pass

    def compute(b, d3):
        xb, pb_ref = xw_bufs[b], p_bufs[b]

        @pl.loop(0, CHB // 16)
        def _(j):
            sl = pl.ds(j * 16, 16)
            s16 = src_idxs[b][sl]
            d16 = dst_idxs[d3][sl]
            a_s = plsc.load_gather(asd_buf, [s16 * 2])
            a_d = plsc.load_gather(asd_buf, [d16 * 2 + 1])
            e = a_s + a_d
            e = jnp.where(e > 0, e, 0.01 * e)
            p = jnp.exp(e)
            pb_ref[sl] = p
            for kk in range(16):
                plsc.addupdate_scatter(z_buf, [d16], p, mask=lane == kk)

        @pl.loop(0, CHB // 4)
        def _(q):
            r0 = q * 4
            for rr in range(4):
                r = r0 + rr
                pb = plsc.load_gather(pb_ref, [jnp.full((16,), r, jnp.int32)])
                for j in range(H // 16):
                    sl = pl.ds(j * 16, 16)
                    xb[r, sl] = xb[r, sl] * pb

    plsc.subcore_barrier()
    _edge_pipeline(wid, E // CHB, CHB, src_hbm, dst_hbm, xw_hbm,
                   src_idxs, dst_idxs, xw_bufs, s_src, s_dst, s_g, s_sc,
                   issue_aux, wait_aux, compute, xw_bufs, acc)
    plsc.subcore_barrier()
    _flush_acc_slice(sid, cid, acc, out_hbm)
    pltpu.sync_copy(z_buf, outz_hbm.at[cid, sid])


def kernel(x, edge_index, edge_attr, batch, lin1_W, lin1_b, nd1_W, nd2_W,
           nd_bias, gru0_Wih, gru0_Whh, gru0_bih, gru0_bhh, gat_W, gat_asrc,
           gat_adst, gat_b, gru1_Wih, gru1_Whh, gru1_bih, gru1_bhh, gin_W,
           gin_b, lstm_Wih, lstm_Whh, lstm_bih, lstm_bhh, lin2_W, lin2_b):
    f32 = jnp.float32
    src = edge_index[0]
    dst = edge_index[1]

    w1t = lin1_W.T
    b1 = lin1_b.reshape(1, H)
    wxt = nd1_W[:, :H].T
    wet = nd1_W[:, H:].T
    x0, u = pl.pallas_call(
        _tc1a_body,
        grid=(N // NBLK,),
        in_specs=[
            pl.BlockSpec((NBLK, H), lambda i: (i, 0)),
            pl.BlockSpec((H, H), lambda i: (0, 0)),
            pl.BlockSpec((1, H), lambda i: (0, 0)),
            pl.BlockSpec((H, H), lambda i: (0, 0)),
        ],
        out_specs=[
            pl.BlockSpec((NBLK, H), lambda i: (i, 0)),
            pl.BlockSpec((NBLK, H), lambda i: (i, 0)),
        ],
        out_shape=[jax.ShapeDtypeStruct((N, H), f32),
                   jax.ShapeDtypeStruct((N, H), f32)],
    )(x, w1t, b1, wxt)

    v = pl.pallas_call(
        _tc1b_body,
        grid=(E // EBLK,),
        in_specs=[
            pl.BlockSpec((EBLK, ED), lambda i: (i, 0)),
            pl.BlockSpec((ED, H), lambda i: (0, 0)),
        ],
        out_specs=pl.BlockSpec((EBLK, H), lambda i: (i, 0)),
        out_shape=jax.ShapeDtypeStruct((E, H), f32),
    )(edge_attr, wet)

    mesh = plsc.VectorSubcoreMesh(core_axis_name="c", subcore_axis_name="s")
    sc_params = pltpu.CompilerParams()
    if "needs_layout_passes" in pltpu.CompilerParams.__dataclass_fields__:
        sc_params = dataclasses.replace(sc_params, needs_layout_passes=False)
    sc_a = pl.kernel(
        _sc_a_body,
        out_type=jax.ShapeDtypeStruct((2, N, H), f32),
        mesh=mesh,
        scratch_types=[
            pltpu.VMEM((CH,), jnp.int32),
            pltpu.VMEM((CH,), jnp.int32),
            pltpu.VMEM((CH,), jnp.int32),
            pltpu.VMEM((CH,), jnp.int32),
            pltpu.VMEM((CH,), jnp.int32),
            pltpu.VMEM((CH, H), f32),
            pltpu.VMEM((CH, H), f32),
            pltpu.VMEM((CH, H), f32),
            pltpu.VMEM((CH, H), f32),
            pltpu.VMEM((CH, H), f32),
            pltpu.VMEM((CH, H), f32),
            pltpu.VMEM_SHARED((N, H), f32),
            pltpu.SemaphoreType.DMA((2,)),
            pltpu.SemaphoreType.DMA((3,)),
            pltpu.SemaphoreType.DMA((2,)),
            pltpu.SemaphoreType.DMA((2,)),
            pltpu.SemaphoreType.DMA((2,)),
        ],
        compiler_params=sc_params,
    )
    s_part = sc_a(src, dst, u, v)

    w2t = nd2_W.T
    ndb = nd_bias.reshape(1, H)
    wih0 = gru0_Wih.T
    whh0 = gru0_Whh.T
    bih0 = gru0_bih.reshape(1, 3 * H)
    bhh0 = gru0_bhh.reshape(1, 3 * H)
    gwt = gat_W.T
    asd_w = jnp.stack([gat_asrc, gat_adst], axis=1)
    xh, xw, asd = pl.pallas_call(
        _tc2_body,
        grid=(N // NBLK,),
        in_specs=[
            pl.BlockSpec((2, NBLK, H), lambda i: (0, i, 0)),
            pl.BlockSpec((NBLK, H), lambda i: (i, 0)),
            pl.BlockSpec((H, H), lambda i: (0, 0)),
            pl.BlockSpec((1, H), lambda i: (0, 0)),
            pl.BlockSpec((H, 3 * H), lambda i: (0, 0)),
            pl.BlockSpec((H, 3 * H), lambda i: (0, 0)),
            pl.BlockSpec((1, 3 * H), lambda i: (0, 0)),
            pl.BlockSpec((1, 3 * H), lambda i: (0, 0)),
            pl.BlockSpec((H, H), lambda i: (0, 0)),
            pl.BlockSpec((H, 2), lambda i: (0, 0)),
        ],
        out_specs=[
            pl.BlockSpec((NBLK, H), lambda i: (i, 0)),
            pl.BlockSpec((NBLK, H), lambda i: (i, 0)),
            pl.BlockSpec((NBLK, 2), lambda i: (i, 0)),
        ],
        out_shape=[jax.ShapeDtypeStruct((N, H), f32),
                   jax.ShapeDtypeStruct((N, H), f32),
                   jax.ShapeDtypeStruct((N, 2), f32)],
    )(s_part, x0, w2t, ndb, wih0, whh0, bih0, bhh0, gwt, asd_w)

    asd_flat = asd.reshape(2 * N)
    sc_b = pl.kernel(
        _sc_b_body,
        out_type=[jax.ShapeDtypeStruct((2, N, H), f32),
                  jax.ShapeDtypeStruct((2, 16, N), f32)],
        mesh=mesh,
        scratch_types=[
            pltpu.VMEM((CHB,), jnp.int32),
            pltpu.VMEM((CHB,), jnp.int32),
            pltpu.VMEM((CHB,), jnp.int32),
            pltpu.VMEM((CHB,), jnp.int32),
            pltpu.VMEM((CHB,), jnp.int32),
            pltpu.VMEM((CHB,), f32),
            pltpu.VMEM((CHB,), f32),
            pltpu.VMEM((CHB, H), f32),
            pltpu.VMEM((CHB, H), f32),
            pltpu.VMEM((2 * N,), f32),
            pltpu.VMEM((N,), f32),
            pltpu.VMEM_SHARED((N, H), f32),
            pltpu.SemaphoreType.DMA((2,)),
            pltpu.SemaphoreType.DMA((3,)),
            pltpu.SemaphoreType.DMA((2,)),
            pltpu.SemaphoreType.DMA((2,)),
        ],
        compiler_params=sc_params,
    )
    gat_part, z_part = sc_b(src, dst, xw, asd_flat)

    gatb = gat_b.reshape(1, H)
    wih1 = gru1_Wih.T
    whh1 = gru1_Whh.T
    bih1 = gru1_bih.reshape(1, 3 * H)
    bhh1 = gru1_bhh.reshape(1, 3 * H)
    ginwt = gin_W.T
    ginb = gin_b.reshape(1, H)
    lwih = lstm_Wih.T
    lwhh = lstm_Whh.T
    lbih = lstm_bih.reshape(1, 4 * H)
    lbhh = lstm_bhh.reshape(1, 4 * H)
    l2wt = lin2_W.T
    l2b = lin2_b.reshape(1, H)
    batch3 = batch.reshape(N // NBLK, 1, NBLK)
    out = pl.pallas_call(
        _tc3_body,
        grid=(N // NBLK,),
        in_specs=[
            pl.BlockSpec((2, NBLK, H), lambda i: (0, i, 0)),
            pl.BlockSpec((1, 32, NBLK), lambda i: (i, 0, 0)),
            pl.BlockSpec((NBLK, H), lambda i: (i, 0)),
            pl.BlockSpec((1, 1, NBLK), lambda i: (i, 0, 0)),
            pl.BlockSpec((1, H), lambda i: (0, 0)),
            pl.BlockSpec((H, 3 * H), lambda i: (0, 0)),
            pl.BlockSpec((H, 3 * H), lambda i: (0, 0)),
            pl.BlockSpec((1, 3 * H), lambda i: (0, 0)),
            pl.BlockSpec((1, 3 * H), lambda i: (0, 0)),
            pl.BlockSpec((H, H), lambda i: (0, 0)),
            pl.BlockSpec((1, H), lambda i: (0, 0)),
            pl.BlockSpec((H, 4 * H), lambda i: (0, 0)),
            pl.BlockSpec((H, 4 * H), lambda i: (0, 0)),
            pl.BlockSpec((1, 4 * H), lambda i: (0, 0)),
            pl.BlockSpec((1, 4 * H), lambda i: (0, 0)),
            pl.BlockSpec((H, H), lambda i: (0, 0)),
            pl.BlockSpec((1, H), lambda i: (0, 0)),
        ],
        out_specs=pl.BlockSpec((B, H), lambda i: (0, 0)),
        out_shape=jax.ShapeDtypeStruct((B, H), f32),
        scratch_shapes=[pltpu.VMEM((B, H), f32)],
    )(gat_part, z_part.reshape(32, N // NBLK, NBLK).transpose(1, 0, 2), xh,
      batch3, gatb, wih1, whh1, bih1, bhh1, ginwt, ginb, lwih, lwhh, lbih,
      lbhh, l2wt, l2b)
    return out


# speedup vs baseline: 2.3706x; 1.0112x over previous
import dataclasses
import functools

import jax
import jax.numpy as jnp
from jax import lax
from jax.experimental import pallas as pl
from jax.experimental.pallas import tpu as pltpu
from jax.experimental.pallas import tpu_sc as plsc

N = 10000
E = 320000
H = 128
ED = 16
B = 64
T = 2

NBLK = 2000
EBLK = 16000
CH = 64
CHB = 64
NW = 32
RPT = 624


import numpy as _np

_pi = _np.arange(H)
_pg, _pj = _pi // 32, _pi % 32
QPERM = _np.where(_pj % 2 == 0, 32 * _pg + _pj // 2, 32 * _pg + 16 + _pj // 2)


def _leaky(v):
    return jnp.where(v > 0, v, 0.01 * v)


def _elu(v):
    return jnp.where(v > 0, v, jnp.exp(v) - 1.0)


def _tc1a_body(x_ref, w1t_ref, b1_ref, wxt_ref, x0_ref, u_ref):
    x0 = _leaky(jnp.dot(x_ref[...], w1t_ref[...],
                        preferred_element_type=jnp.float32) + b1_ref[...])
    x0_ref[...] = x0
    u_ref[...] = jnp.dot(x0, wxt_ref[...], preferred_element_type=jnp.float32)


def _tc1b_body(ea_ref, wet_ref, v_ref):
    v_ref[...] = jnp.dot(ea_ref[...], wet_ref[...],
                         preferred_element_type=jnp.float32)


def _tc2_body(sp_ref, x0_ref, w2t_ref, ndb_ref, wih_ref, whh_ref, bih_ref,
              bhh_ref, gwt_ref, asd_w_ref, xh_ref, xw_ref, asd_ref):
    sp = sp_ref[...]
    s = sp[0] + sp[1]
    h = _elu(jnp.dot(s, w2t_ref[...], preferred_element_type=jnp.float32)
             + ndb_ref[...])
    x0 = x0_ref[...]
    gi = jnp.dot(h, wih_ref[...], preferred_element_type=jnp.float32) + bih_ref[...]
    gh = jnp.dot(x0, whh_ref[...], preferred_element_type=jnp.float32) + bhh_ref[...]
    r = jax.nn.sigmoid(gi[:, :H] + gh[:, :H])
    z = jax.nn.sigmoid(gi[:, H:2 * H] + gh[:, H:2 * H])
    n = jnp.tanh(gi[:, 2 * H:] + r * gh[:, 2 * H:])
    xh = jax.nn.relu((1.0 - z) * n + z * x0)
    xh_ref[...] = xh
    xw = jnp.dot(xh, gwt_ref[...], preferred_element_type=jnp.float32)
    xw_ref[...] = xw
    asd_ref[...] = jnp.dot(xw, asd_w_ref[...], preferred_element_type=jnp.float32)


def _tc3_body(acc_ref, zp_ref, xh_ref, batch_ref, gatb_ref, wih_ref, whh_ref,
              bih_ref, bhh_ref, ginwt_ref, ginb_ref, lwih_ref, lwhh_ref,
              lbih_ref, lbhh_ref, l2wt_ref, l2b_ref, out_ref, pooled_ref):
    i = pl.program_id(0)
    acc = acc_ref[...]
    w = acc[0] + acc[1]
    zp = zp_ref[...][0]
    z = jnp.sum(zp, axis=0)[:, None]
    gat = w / (z + 1e-16) + gatb_ref[...]
    h2 = _elu(gat)
    xh = xh_ref[...]
    gi = jnp.dot(h2, wih_ref[...], preferred_element_type=jnp.float32) + bih_ref[...]
    gh = jnp.dot(xh, whh_ref[...], preferred_element_type=jnp.float32) + bhh_ref[...]
    r = jax.nn.sigmoid(gi[:, :H] + gh[:, :H])
    zz = jax.nn.sigmoid(gi[:, H:2 * H] + gh[:, H:2 * H])
    n = jnp.tanh(gi[:, 2 * H:] + r * gh[:, 2 * H:])
    acts = jax.nn.relu((1.0 - zz) * n + zz * xh)

    bb = batch_ref[...][0]
    seg = lax.broadcasted_iota(jnp.int32, (B, NBLK), 0)
    oh = (bb == seg).astype(jnp.float32)
    part = jnp.dot(oh, acts, preferred_element_type=jnp.float32)

    @pl.when(i == 0)
    def _():
        pooled_ref[...] = part

    @pl.when(i > 0)
    def _():
        pooled_ref[...] = pooled_ref[...] + part

    @pl.when(i == pl.num_programs(0) - 1)
    def _():
        pooled = pooled_ref[...]
        out = jax.nn.relu(pooled)
        for _ in range(T):
            ghh = _elu(jnp.dot(out + pooled, ginwt_ref[...],
                               preferred_element_type=jnp.float32) + ginb_ref[...])
            g = (jnp.dot(out, lwih_ref[...], preferred_element_type=jnp.float32)
                 + lbih_ref[...]
                 + jnp.dot(ghh, lwhh_ref[...], preferred_element_type=jnp.float32)
                 + lbhh_ref[...])
            ig = jax.nn.sigmoid(g[:, :H])
            fg = jax.nn.sigmoid(g[:, H:2 * H])
            gg = jnp.tanh(g[:, 2 * H:3 * H])
            og = jax.nn.sigmoid(g[:, 3 * H:])
            c2 = fg * ghh + ig * gg
            out = og * jnp.tanh(c2)
        out_ref[...] = (jnp.dot(out, l2wt_ref[...],
                                preferred_element_type=jnp.float32) + l2b_ref[...])


def _zero_rows(buf, nrows):
    zero16 = jnp.zeros((16,), jnp.float32)
    ncols = buf.shape[1]

    @pl.loop(0, nrows)
    def _(r):
        for j in range(ncols // 16):
            buf[r, pl.ds(j * 16, 16)] = zero16


def _zero_acc_slice(sid, zbuf, acc):
    base_r = sid * RPT
    zch = zbuf.shape[0]
    for off in range(0, RPT, zch):
        sz = min(zch, RPT - off)
        pltpu.sync_copy(zbuf.at[pl.ds(0, sz)], acc.at[pl.ds(base_r + off, sz)])

    @pl.when(sid == 15)
    def _():
        pltpu.sync_copy(zbuf.at[pl.ds(0, N - 16 * RPT)],
                        acc.at[pl.ds(16 * RPT, N - 16 * RPT)])


def _flush_acc_slice(sid, cid, acc, out_hbm):
    base_r = sid * RPT
    pltpu.sync_copy(acc.at[pl.ds(base_r, RPT)],
                    out_hbm.at[cid, pl.ds(base_r, RPT)])

    @pl.when(sid == 15)
    def _():
        pltpu.sync_copy(acc.at[pl.ds(16 * RPT, N - 16 * RPT)],
                        out_hbm.at[cid, pl.ds(16 * RPT, N - 16 * RPT)])


def _edge_pipeline(wid, nchunk, ch, src_hbm, dst_hbm, table_hbm,
                   src_idxs, dst_idxs, g_bufs, s_src, s_dst, s_g, s_sc,
                   issue_aux, wait_aux, compute, scat_bufs, acc):
    def issue_front(c_idx, gb, d3):
        ebase = c_idx * ch
        pltpu.async_copy(table_hbm.at[src_idxs[gb]], g_bufs[gb], s_g.at[gb])
        pltpu.async_copy(dst_hbm.at[pl.ds(ebase, ch)], dst_idxs[d3],
                         s_dst.at[d3])
        issue_aux(gb, ebase)

    pltpu.sync_copy(src_hbm.at[pl.ds(wid * ch, ch)], src_idxs[0])
    issue_front(wid, 0, 0)
    pltpu.async_copy(src_hbm.at[pl.ds((wid + NW) * ch, ch)], src_idxs[1],
                     s_src.at[1])

    itmax = -(-nchunk // NW)
    k6 = (itmax + 6) // 6

    @pl.loop(0, k6)
    def _(k):
        for h in range(6):
            it = k * 6 + h
            b, bn = h % 2, 1 - h % 2
            d3, d3n, d3p = h % 3, (h + 1) % 3, (h + 2) % 3
            c_idx = wid + it * NW

            @pl.when(c_idx + NW < nchunk)
            def _():
                pltpu.make_async_copy(src_hbm.at[pl.ds(0, ch)], src_idxs[bn],
                                      s_src.at[bn]).wait()
                issue_front(c_idx + NW, bn, d3n)

            @pl.when(c_idx < nchunk)
            def _():
                pltpu.make_async_copy(dst_hbm.at[pl.ds(0, ch)], dst_idxs[d3],
                                      s_dst.at[d3]).wait()
                pltpu.make_async_copy(table_hbm.at[src_idxs[b]], g_bufs[b],
                                      s_g.at[b]).wait()
                wait_aux(b)
                compute(b, d3)

                @pl.when(it >= 1)
                def _():
                    pltpu.make_async_copy(scat_bufs[bn], acc.at[dst_idxs[d3p]],
                                          s_sc.at[bn]).wait()

                pltpu.async_copy(scat_bufs[b], acc.at[dst_idxs[d3]],
                                 s_sc.at[b], add=True)

            @pl.when((c_idx >= nchunk) & (c_idx < nchunk + NW) & (it >= 1))
            def _():
                pltpu.make_async_copy(scat_bufs[bn], acc.at[dst_idxs[d3p]],
                                      s_sc.at[bn]).wait()

            @pl.when(c_idx + 2 * NW < nchunk)
            def _():
                pltpu.async_copy(src_hbm.at[pl.ds((c_idx + 2 * NW) * ch, ch)],
                                 src_idxs[b], s_src.at[b])


def _sc_a_body(src_hbm, dst_hbm, u_hbm, v_hbm, out_hbm,
               src_idx0, src_idx1, dst_idx0, dst_idx1, dst_idx2, u0, u1,
               v0, v1, m0, m1, acc, s_src, s_dst, s_g, s_aux, s_sc):
    cid = lax.axis_index("c")
    sid = lax.axis_index("s")
    wid = sid * 2 + cid
    src_idxs = (src_idx0, src_idx1)
    dst_idxs = (dst_idx0, dst_idx1, dst_idx2)
    u_bufs, v_bufs, m_bufs = (u0, u1), (v0, v1), (m0, m1)

    _zero_rows(m0, CH)
    _zero_acc_slice(sid, m0, acc)

    def issue_aux(bn, ebase):
        pltpu.async_copy(v_hbm.at[pl.ds(ebase, CH)], v_bufs[bn], s_aux.at[bn])

    def wait_aux(b):
        pltpu.make_async_copy(v_hbm.at[pl.ds(0, CH)], v_bufs[b],
                              s_aux.at[b]).wait()

    def compute(b, d3):
        ub, vb, mb = u_bufs[b], v_bufs[b], m_bufs[b]

        @pl.loop(0, CH // 4)
        def _(q):
            r0 = q * 4
            for rr in range(4):
                r = r0 + rr
                for j in range(H // 16):
                    sl = pl.ds(j * 16, 16)
                    t = ub[r, sl] + vb[r, sl]
                    mb[r, sl] = jnp.where(t > 0, t, 0.01 * t)

    plsc.subcore_barrier()
    _edge_pipeline(wid, E // CH, CH, src_hbm, dst_hbm, u_hbm,
                   src_idxs, dst_idxs, u_bufs, s_src, s_dst, s_g, s_sc,
                   issue_aux, wait_aux, compute, m_bufs, acc)
    plsc.subcore_barrier()
    _flush_acc_slice(sid, cid, acc, out_hbm)


def _sc_b_body(src_hbm, dst_hbm, xw_hbm, asd_hbm, out_hbm, outz_hbm,
               src_idx0, src_idx1, dst_idx0, dst_idx1, dst_idx2, pb0, pb1,
               xw0, xw1, asd_buf, z_buf, acc, s_src, s_dst, s_g, s_sc):
    cid = lax.axis_index("c")
    sid = lax.axis_index("s")
    wid = sid * 2 + cid
    lane = lax.iota(jnp.int32, 16)
    zero16 = jnp.zeros((16,), jnp.float32)
    src_idxs = (src_idx0, src_idx1)
    dst_idxs = (dst_idx0, dst_idx1, dst_idx2)
    p_bufs, xw_bufs = (pb0, pb1), (xw0, xw1)

    _zero_rows(xw0, CHB)
    _zero_acc_slice(sid, xw0, acc)

    @pl.loop(0, N // 16)
    def _(r):
        z_buf[pl.ds(r * 16, 16)] = zero16

    pltpu.sync_copy(asd_hbm, asd_buf)

    def issue_aux(bn, ebase):
        pass

    def wait_aux(b):
        pass

    def compute(b, d3):
        xb, pb_ref = xw_bufs[b], p_bufs[b]

        @pl.loop(0, CHB // 16)
        def _(j):
            sl = pl.ds(j * 16, 16)
            s16 = src_idxs[b][sl]
            d16 = dst_idxs[d3][sl]
            a_s = plsc.load_gather(asd_buf, [s16 * 2])
            a_d = plsc.load_gather(asd_buf, [d16 * 2 + 1])
            e = a_s + a_d
            e = jnp.where(e > 0, e, 0.01 * e)
            p = jnp.exp(e)
            pb_ref[sl] = p
            for kk in range(16):
                plsc.addupdate_scatter(z_buf, [d16], p, mask=lane == kk)

        @pl.loop(0, CHB // 4)
        def _(q):
            r0 = q * 4
            for rr in range(4):
                r = r0 + rr
                pb = plsc.load_gather(pb_ref, [jnp.full((16,), r, jnp.int32)])
                for j in range(H // 16):
                    sl = pl.ds(j * 16, 16)
                    xb[r, sl] = xb[r, sl] * pb

    plsc.subcore_barrier()
    _edge_pipeline(wid, E // CHB, CHB, src_hbm, dst_hbm, xw_hbm,
                   src_idxs, dst_idxs, xw_bufs, s_src, s_dst, s_g, s_sc,
                   issue_aux, wait_aux, compute, xw_bufs, acc)
    plsc.subcore_barrier()
    _flush_acc_slice(sid, cid, acc, out_hbm)
    pltpu.sync_copy(z_buf, outz_hbm.at[cid, sid])


def kernel(x, edge_index, edge_attr, batch, lin1_W, lin1_b, nd1_W, nd2_W,
           nd_bias, gru0_Wih, gru0_Whh, gru0_bih, gru0_bhh, gat_W, gat_asrc,
           gat_adst, gat_b, gru1_Wih, gru1_Whh, gru1_bih, gru1_bhh, gin_W,
           gin_b, lstm_Wih, lstm_Whh, lstm_bih, lstm_bhh, lin2_W, lin2_b):
    f32 = jnp.float32
    src = edge_index[0]
    dst = edge_index[1]

    w1t = lin1_W.T
    b1 = lin1_b.reshape(1, H)
    wxt = nd1_W[:, :H].T
    wet = nd1_W[:, H:].T
    x0, u = pl.pallas_call(
        _tc1a_body,
        grid=(N // NBLK,),
        in_specs=[
            pl.BlockSpec((NBLK, H), lambda i: (i, 0)),
            pl.BlockSpec((H, H), lambda i: (0, 0)),
            pl.BlockSpec((1, H), lambda i: (0, 0)),
            pl.BlockSpec((H, H), lambda i: (0, 0)),
        ],
        out_specs=[
            pl.BlockSpec((NBLK, H), lambda i: (i, 0)),
            pl.BlockSpec((NBLK, H), lambda i: (i, 0)),
        ],
        out_shape=[jax.ShapeDtypeStruct((N, H), f32),
                   jax.ShapeDtypeStruct((N, H), f32)],
    )(x, w1t, b1, wxt)

    v = pl.pallas_call(
        _tc1b_body,
        grid=(E // EBLK,),
        in_specs=[
            pl.BlockSpec((EBLK, ED), lambda i: (i, 0)),
            pl.BlockSpec((ED, H), lambda i: (0, 0)),
        ],
        out_specs=pl.BlockSpec((EBLK, H), lambda i: (i, 0)),
        out_shape=jax.ShapeDtypeStruct((E, H), f32),
    )(edge_attr, wet)

    mesh = plsc.VectorSubcoreMesh(core_axis_name="c", subcore_axis_name="s")
    sc_params = pltpu.CompilerParams()
    if "needs_layout_passes" in pltpu.CompilerParams.__dataclass_fields__:
        sc_params = dataclasses.replace(sc_params, needs_layout_passes=False)
    sc_a = pl.kernel(
        _sc_a_body,
        out_type=jax.ShapeDtypeStruct((2, N, H), f32),
        mesh=mesh,
        scratch_types=[
            pltpu.VMEM((CH,), jnp.int32),
            pltpu.VMEM((CH,), jnp.int32),
            pltpu.VMEM((CH,), jnp.int32),
            pltpu.VMEM((CH,), jnp.int32),
            pltpu.VMEM((CH,), jnp.int32),
            pltpu.VMEM((CH, H), f32),
            pltpu.VMEM((CH, H), f32),
            pltpu.VMEM((CH, H), f32),
            pltpu.VMEM((CH, H), f32),
            pltpu.VMEM((CH, H), f32),
            pltpu.VMEM((CH, H), f32),
            pltpu.VMEM_SHARED((N, H), f32),
            pltpu.SemaphoreType.DMA((2,)),
            pltpu.SemaphoreType.DMA((3,)),
            pltpu.SemaphoreType.DMA((2,)),
            pltpu.SemaphoreType.DMA((2,)),
            pltpu.SemaphoreType.DMA((2,)),
        ],
        compiler_params=sc_params,
    )
    s_part = sc_a(src, dst, u, v)

    w2t = nd2_W.T
    ndb = nd_bias.reshape(1, H)
    wih0 = gru0_Wih.T
    whh0 = gru0_Whh.T
    bih0 = gru0_bih.reshape(1, 3 * H)
    bhh0 = gru0_bhh.reshape(1, 3 * H)
    gwt = gat_W.T
    asd_w = jnp.stack([gat_asrc, gat_adst], axis=1)
    xh, xw, asd = pl.pallas_call(
        _tc2_body,
        grid=(N // NBLK,),
        in_specs=[
            pl.BlockSpec((2, NBLK, H), lambda i: (0, i, 0)),
            pl.BlockSpec((NBLK, H), lambda i: (i, 0)),
            pl.BlockSpec((H, H), lambda i: (0, 0)),
            pl.BlockSpec((1, H), lambda i: (0, 0)),
            pl.BlockSpec((H, 3 * H), lambda i: (0, 0)),
            pl.BlockSpec((H, 3 * H), lambda i: (0, 0)),
            pl.BlockSpec((1, 3 * H), lambda i: (0, 0)),
            pl.BlockSpec((1, 3 * H), lambda i: (0, 0)),
            pl.BlockSpec((H, H), lambda i: (0, 0)),
            pl.BlockSpec((H, 2), lambda i: (0, 0)),
        ],
        out_specs=[
            pl.BlockSpec((NBLK, H), lambda i: (i, 0)),
            pl.BlockSpec((NBLK, H), lambda i: (i, 0)),
            pl.BlockSpec((NBLK, 2), lambda i: (i, 0)),
        ],
        out_shape=[jax.ShapeDtypeStruct((N, H), f32),
                   jax.ShapeDtypeStruct((N, H), f32),
                   jax.ShapeDtypeStruct((N, 2), f32)],
    )(s_part, x0, w2t, ndb, wih0, whh0, bih0, bhh0, gwt, asd_w)

    asd_flat = asd.reshape(2 * N)
    sc_b = pl.kernel(
        _sc_b_body,
        out_type=[jax.ShapeDtypeStruct((2, N, H), f32),
                  jax.ShapeDtypeStruct((2, 16, N), f32)],
        mesh=mesh,
        scratch_types=[
            pltpu.VMEM((CHB,), jnp.int32),
            pltpu.VMEM((CHB,), jnp.int32),
            pltpu.VMEM((CHB,), jnp.int32),
            pltpu.VMEM((CHB,), jnp.int32),
            pltpu.VMEM((CHB,), jnp.int32),
            pltpu.VMEM((CHB,), f32),
            pltpu.VMEM((CHB,), f32),
            pltpu.VMEM((CHB, H), f32),
            pltpu.VMEM((CHB, H), f32),
            pltpu.VMEM((2 * N,), f32),
            pltpu.VMEM((N,), f32),
            pltpu.VMEM_SHARED((N, H), f32),
            pltpu.SemaphoreType.DMA((2,)),
            pltpu.SemaphoreType.DMA((3,)),
            pltpu.SemaphoreType.DMA((2,)),
            pltpu.SemaphoreType.DMA((2,)),
        ],
        compiler_params=sc_params,
    )
    gat_part, z_part = sc_b(src, dst, xw, asd_flat)

    gatb = gat_b.reshape(1, H)
    wih1 = gru1_Wih.T
    whh1 = gru1_Whh.T
    bih1 = gru1_bih.reshape(1, 3 * H)
    bhh1 = gru1_bhh.reshape(1, 3 * H)
    ginwt = gin_W.T
    ginb = gin_b.reshape(1, H)
    lwih = lstm_Wih.T
    lwhh = lstm_Whh.T
    lbih = lstm_bih.reshape(1, 4 * H)
    lbhh = lstm_bhh.reshape(1, 4 * H)
    l2wt = lin2_W.T
    l2b = lin2_b.reshape(1, H)
    batch3 = batch.reshape(N // NBLK, 1, NBLK)
    out = pl.pallas_call(
        _tc3_body,
        grid=(N // NBLK,),
        in_specs=[
            pl.BlockSpec((2, NBLK, H), lambda i: (0, i, 0)),
            pl.BlockSpec((1, 32, NBLK), lambda i: (i, 0, 0)),
            pl.BlockSpec((NBLK, H), lambda i: (i, 0)),
            pl.BlockSpec((1, 1, NBLK), lambda i: (i, 0, 0)),
            pl.BlockSpec((1, H), lambda i: (0, 0)),
            pl.BlockSpec((H, 3 * H), lambda i: (0, 0)),
            pl.BlockSpec((H, 3 * H), lambda i: (0, 0)),
            pl.BlockSpec((1, 3 * H), lambda i: (0, 0)),
            pl.BlockSpec((1, 3 * H), lambda i: (0, 0)),
            pl.BlockSpec((H, H), lambda i: (0, 0)),
            pl.BlockSpec((1, H), lambda i: (0, 0)),
            pl.BlockSpec((H, 4 * H), lambda i: (0, 0)),
            pl.BlockSpec((H, 4 * H), lambda i: (0, 0)),
            pl.BlockSpec((1, 4 * H), lambda i: (0, 0)),
            pl.BlockSpec((1, 4 * H), lambda i: (0, 0)),
            pl.BlockSpec((H, H), lambda i: (0, 0)),
            pl.BlockSpec((1, H), lambda i: (0, 0)),
        ],
        out_specs=pl.BlockSpec((B, H), lambda i: (0, 0)),
        out_shape=jax.ShapeDtypeStruct((B, H), f32),
        scratch_shapes=[pltpu.VMEM((B, H), f32)],
    )(gat_part, z_part.reshape(32, N // NBLK, NBLK).transpose(1, 0, 2), xh,
      batch3, gatb, wih1, whh1, bih1, bhh1, ginwt, ginb, lwih, lwhh, lbih,
      lbhh, l2wt, l2b)
    return out
